# baseline (speedup 1.0000x reference)
import functools

import jax
import jax.numpy as jnp
from jax import lax
from jax.experimental import pallas as pl
from jax.experimental.pallas import tpu as pltpu
from jax.experimental.pallas import tpu_sc as plsc

X_MIN, Y_MIN, Z_MIN = -51.2, -51.2, -5.0
X_MAX, Y_MAX, Z_MAX = 51.2, 51.2, 3.0
VX, VY = 0.4, 0.4
XS = int((X_MAX - X_MIN) / VX)
YS = int((Y_MAX - Y_MIN) / VY)
NCELL = YS * XS
BN_EPS = 1e-5


NB = 4
NPTS = 20000
NWIN = NCELL + 16
_ACH = 2000
_BCH = 1024


def _vox_body(pts_ref, neg1_ref, out_ref, shared_win):
    c = lax.axis_index("c")
    s = lax.axis_index("s")
    lane = lax.iota(jnp.int32, 16)
    row = s // 8
    b = c * 2 + row
    pbase = b * 5 * NPTS

    def phase_a(win, xb, yb, zb):
        pltpu.sync_copy(neg1_ref, win)
        for ci in range(NPTS // _ACH):
            for buf, f in ((xb, 0), (yb, 1), (zb, 2)):
                off = pl.multiple_of(pbase + f * NPTS + ci * _ACH, 8)
                pltpu.sync_copy(pts_ref.at[pl.ds(off, _ACH)], buf)

            def step(i, _):
                o = pl.multiple_of(i * 16, 16)
                xv = xb[pl.ds(o, 16)]
                yv = yb[pl.ds(o, 16)]
                zv = zb[pl.ds(o, 16)]
                valid = ((xv >= X_MIN) & (xv < X_MAX) &
                         (yv >= Y_MIN) & (yv < Y_MAX) &
                         (zv >= Z_MIN) & (zv < Z_MAX))
                xi = jnp.clip(((xv - X_MIN) / VX).astype(jnp.int32), 0, XS - 1)
                yi = jnp.clip(((yv - Y_MIN) / VY).astype(jnp.int32), 0, YS - 1)
                vox = jnp.where(valid, yi * XS + xi, NCELL + lane)
                pidx = ci * _ACH + i * 16 + lane
                plsc.store_scatter(win, [vox], pidx)
                return _

            lax.fori_loop(0, _ACH // 16, step, None)
        woff = pl.multiple_of(row * NWIN, 8)
        pltpu.sync_copy(win, shared_win.at[pl.ds(woff, NWIN)])

    @pl.when((s == 0) | (s == 8))
    def _():
        pl.run_scoped(phase_a,
                      pltpu.VMEM((NWIN,), jnp.int32),
                      pltpu.VMEM((_ACH,), jnp.float32),
                      pltpu.VMEM((_ACH,), jnp.float32),
                      pltpu.VMEM((_ACH,), jnp.float32))

    plsc.subcore_barrier()

    sl = s % 8
    cells0 = sl * (NCELL // 8)

    def phase_b(fbuf, wsl, obuf):
        pltpu.sync_copy(pts_ref.at[pl.ds(pl.multiple_of(pbase, 8), 5 * NPTS)],
                        fbuf)
        pltpu.sync_copy(
            shared_win.at[pl.ds(pl.multiple_of(row * NWIN + cells0, 8),
                                NCELL // 8)], wsl)
        for k in range(NCELL // 8 // _BCH):

            def step(i, _):
                o = pl.multiple_of(k * _BCH + i * 16, 16)
                w = wsl[pl.ds(o, 16)]
                valid = w >= 0
                wsafe = jnp.where(valid, w, lane)
                oo = pl.multiple_of(i * 16, 16)
                for f in range(5):
                    v = plsc.load_gather(fbuf, [wsafe + f * NPTS])
                    obuf[pl.ds(oo + f * _BCH, 16)] = jnp.where(valid, v, 0.0)
                return _

            lax.fori_loop(0, _BCH // 16, step, None)
            for f in range(5):
                dst = pl.multiple_of((b * 5 + f) * NCELL + cells0 + k * _BCH, 8)
                pltpu.sync_copy(
                    obuf.at[pl.ds(pl.multiple_of(f * _BCH, 8), _BCH)],
                    out_ref.at[pl.ds(dst, _BCH)])

    pl.run_scoped(phase_b,
                  pltpu.VMEM((5 * NPTS,), jnp.float32),
                  pltpu.VMEM((NCELL // 8,), jnp.int32),
                  pltpu.VMEM((5 * _BCH,), jnp.float32))


def _voxelize(radar):
    pts_t = radar.transpose(0, 2, 1).reshape(NB * 5 * NPTS)
    neg1 = jnp.full((NWIN,), -1, jnp.int32)
    mesh = plsc.VectorSubcoreMesh(core_axis_name="c", subcore_axis_name="s")
    grid = pl.kernel(
        _vox_body,
        mesh=mesh,
        out_type=jax.ShapeDtypeStruct((NB * 5 * NCELL,), jnp.float32),
        scratch_types=[pltpu.VMEM_SHARED((2 * NWIN,), jnp.int32)],
        compiler_params=pltpu.CompilerParams(needs_layout_passes=False),
    )(pts_t, neg1)
    return grid.reshape(NB, 5, YS, XS)


def _shift_w(x, dx):
    c, r = x.shape[0], x.shape[1]
    zcol = jnp.zeros((c, r, 1), jnp.float32)
    if dx == 0:
        return jnp.concatenate([zcol, x[:, :, :XS - 1]], axis=2)
    if dx == 1:
        return x
    return jnp.concatenate([x[:, :, 1:], zcol], axis=2)


_CK = 32


def _conv_body(x_ref, stats_ref, w_ref, b_ref, g_ref, be_ref,
               yc_ref, ostats_ref, xp_scr, stage_scr, sem, *,
               cin, cout, norm_in, rows):
    bi = pl.program_id(0)
    t = pl.program_id(1)

    @pl.when(t == 0)
    def _prologue():
        if norm_in:
            s1 = stats_ref[0, 0][:, None, None]
            s2 = stats_ref[0, 1][:, None, None]
            m = s1 / NCELL
            v = s2 / NCELL - m * m
            a = g_ref[...][:, :, None] * jax.lax.rsqrt(v + BN_EPS)
            c = be_ref[...][:, :, None] - m * a

        def _copy(i, buf):
            return pltpu.make_async_copy(
                x_ref.at[bi, :, pl.ds(i * _CK, _CK), :],
                stage_scr.at[buf], sem.at[buf])

        nck = YS // _CK
        _copy(0, 0).start()
        _copy(1, 1).start()
        for i in range(nck):
            _copy(i, i % 2).wait()
            xc = stage_scr[i % 2]
            if norm_in:
                xc = jnp.maximum(a * xc + c, 0.0)
            xp_scr[:, 8 + i * _CK:8 + (i + 1) * _CK, :] = xc
            if i + 2 < nck:
                _copy(i + 2, i % 2).start()
        xp_scr[:, 0:8, :] = jnp.zeros((cin, 8, XS), jnp.float32)
        xp_scr[:, YS + 8:YS + 16, :] = jnp.zeros((cin, 8, XS), jnp.float32)

    xt = xp_scr[:, pl.ds(t * rows, rows + 16), :]
    xcat = jnp.concatenate(
        [xt[:, 7 + dy:7 + dy + rows, :] for dy in range(3)], axis=0)
    acc = jnp.zeros((cout, rows * XS), jnp.float32)
    for dx in range(3):
        xs = _shift_w(xcat, dx).reshape(3 * cin, rows * XS)
        acc = acc + jax.lax.dot_general(
            w_ref[dx], xs, (((1,), (0,)), ((), ())),
            preferred_element_type=jnp.float32)
    acc = acc + b_ref[...]
    yc_ref[...] = acc.reshape(1, cout, rows, XS)

    @pl.when(t == 0)
    def _init_stats():
        ostats_ref[...] = jnp.zeros((1, 2, cout), jnp.float32)

    ostats_ref[0, 0] = ostats_ref[0, 0] + jnp.sum(acc, axis=1)
    ostats_ref[0, 1] = ostats_ref[0, 1] + jnp.sum(acc * acc, axis=1)


def _conv_layer(x, stats, w3, b, g, be, norm_in, rows=16):
    bsz, cin = x.shape[0], x.shape[1]
    cout = w3.shape[1]
    nt = YS // rows
    body = functools.partial(_conv_body, cin=cin, cout=cout,
                             norm_in=norm_in, rows=rows)
    return pl.pallas_call(
        body,
        grid=(bsz, nt),
        in_specs=[
            pl.BlockSpec(memory_space=pl.ANY),
            pl.BlockSpec((1, 2, cin), lambda i, t: (i, 0, 0)),
            pl.BlockSpec((3, cout, 3 * cin), lambda i, t: (0, 0, 0)),
            pl.BlockSpec((cout, 1), lambda i, t: (0, 0)),
            pl.BlockSpec((cin, 1), lambda i, t: (0, 0)),
            pl.BlockSpec((cin, 1), lambda i, t: (0, 0)),
        ],
        out_specs=[
            pl.BlockSpec((1, cout, rows, XS), lambda i, t: (i, 0, t, 0)),
            pl.BlockSpec((1, 2, cout), lambda i, t: (i, 0, 0)),
        ],
        out_shape=[
            jax.ShapeDtypeStruct((bsz, cout, YS, XS), jnp.float32),
            jax.ShapeDtypeStruct((bsz, 2, cout), jnp.float32),
        ],
        scratch_shapes=[
            pltpu.VMEM((cin, YS + 16, XS), jnp.float32),
            pltpu.VMEM((2, cin, _CK, XS), jnp.float32),
            pltpu.SemaphoreType.DMA((2,)),
        ],
    )(x, stats, w3, b, g, be)


def _final_body(y_ref, stats_ref, g_ref, be_ref, o_ref):
    s1 = stats_ref[0, 0][:, None, None]
    s2 = stats_ref[0, 1][:, None, None]
    m = s1 / NCELL
    v = s2 / NCELL - m * m
    a = g_ref[...][:, :, None] * jax.lax.rsqrt(v + BN_EPS)
    c = be_ref[...][:, :, None] - m * a
    o_ref[0] = jnp.maximum(a * y_ref[0] + c, 0.0)


def _final_norm(y, stats, g, be, rows=64):
    bsz, cout = y.shape[0], y.shape[1]
    return pl.pallas_call(
        _final_body,
        grid=(bsz, YS // rows),
        in_specs=[
            pl.BlockSpec((1, cout, rows, XS), lambda i, t: (i, 0, t, 0)),
            pl.BlockSpec((1, 2, cout), lambda i, t: (i, 0, 0)),
            pl.BlockSpec((cout, 1), lambda i, t: (0, 0)),
            pl.BlockSpec((cout, 1), lambda i, t: (0, 0)),
        ],
        out_specs=pl.BlockSpec((1, cout, rows, XS), lambda i, t: (i, 0, t, 0)),
        out_shape=jax.ShapeDtypeStruct((bsz, cout, YS, XS), jnp.float32),
    )(y, stats, g, be)


def _w3(W):
    return W.transpose(3, 0, 2, 1).reshape(3, W.shape[0], 3 * W.shape[1])


def kernel(radar_points_list, W1, b1, g1, be1, W2, b2, g2, be2,
           W3, b3, g3, be3, W4, b4, g4, be4):
    grid = _voxelize(radar_points_list)
    bsz = grid.shape[0]
    dummy_stats = jnp.zeros((bsz, 2, 5), jnp.float32)
    dummy_gb = jnp.zeros((5, 1), jnp.float32)
    h, s = _conv_layer(grid, dummy_stats, _w3(W1), b1[:, None],
                       dummy_gb, dummy_gb, norm_in=False)
    for (W, b, g, be, gp, bep) in ((W2, b2, g2, be2, g1, be1),
                                   (W3, b3, g3, be3, g2, be2),
                                   (W4, b4, g4, be4, g3, be3)):
        h, s_next = _conv_layer(h, s, _w3(W), b[:, None],
                                gp[:, None], bep[:, None], norm_in=True)
        s = s_next
    return _final_norm(h, s, g4[:, None], be4[:, None])


# speedup vs baseline: 1.0102x; 1.0102x over previous
import functools

import jax
import jax.numpy as jnp
from jax import lax
from jax.experimental import pallas as pl
from jax.experimental.pallas import tpu as pltpu
from jax.experimental.pallas import tpu_sc as plsc

X_MIN, Y_MIN, Z_MIN = -51.2, -51.2, -5.0
X_MAX, Y_MAX, Z_MAX = 51.2, 51.2, 3.0
VX, VY = 0.4, 0.4
XS = int((X_MAX - X_MIN) / VX)
YS = int((Y_MAX - Y_MIN) / VY)
NCELL = YS * XS
BN_EPS = 1e-5


NB = 4
NPTS = 20000
NWIN = NCELL + 16
_ACH = 2000
_BCH = 1024


def _vox_body(pts_ref, neg1_ref, out_ref, shared_win):
    c = lax.axis_index("c")
    s = lax.axis_index("s")
    lane = lax.iota(jnp.int32, 16)
    row = s // 8
    b = c * 2 + row
    pbase = b * 5 * NPTS

    def phase_a(win, xb, yb, zb):
        pltpu.sync_copy(neg1_ref, win)
        for ci in range(NPTS // _ACH):
            for buf, f in ((xb, 0), (yb, 1), (zb, 2)):
                off = pl.multiple_of(pbase + f * NPTS + ci * _ACH, 8)
                pltpu.sync_copy(pts_ref.at[pl.ds(off, _ACH)], buf)

            def step(i, _):
                o = pl.multiple_of(i * 16, 16)
                xv = xb[pl.ds(o, 16)]
                yv = yb[pl.ds(o, 16)]
                zv = zb[pl.ds(o, 16)]
                valid = ((xv >= X_MIN) & (xv < X_MAX) &
                         (yv >= Y_MIN) & (yv < Y_MAX) &
                         (zv >= Z_MIN) & (zv < Z_MAX))
                xi = jnp.clip(((xv - X_MIN) / VX).astype(jnp.int32), 0, XS - 1)
                yi = jnp.clip(((yv - Y_MIN) / VY).astype(jnp.int32), 0, YS - 1)
                vox = jnp.where(valid, yi * XS + xi, NCELL + lane)
                pidx = ci * _ACH + i * 16 + lane
                plsc.store_scatter(win, [vox], pidx)
                return _

            lax.fori_loop(0, _ACH // 16, step, None)
        woff = pl.multiple_of(row * NWIN, 8)
        pltpu.sync_copy(win, shared_win.at[pl.ds(woff, NWIN)])

    @pl.when((s == 0) | (s == 8))
    def _():
        pl.run_scoped(phase_a,
                      pltpu.VMEM((NWIN,), jnp.int32),
                      pltpu.VMEM((_ACH,), jnp.float32),
                      pltpu.VMEM((_ACH,), jnp.float32),
                      pltpu.VMEM((_ACH,), jnp.float32))

    plsc.subcore_barrier()

    sl = s % 8
    cells0 = sl * (NCELL // 8)

    def phase_b(fbuf, wsl, obuf):
        pltpu.sync_copy(pts_ref.at[pl.ds(pl.multiple_of(pbase, 8), 5 * NPTS)],
                        fbuf)
        pltpu.sync_copy(
            shared_win.at[pl.ds(pl.multiple_of(row * NWIN + cells0, 8),
                                NCELL // 8)], wsl)
        for k in range(NCELL // 8 // _BCH):

            def step(i, _):
                o = pl.multiple_of(k * _BCH + i * 16, 16)
                w = wsl[pl.ds(o, 16)]
                valid = w >= 0
                wsafe = jnp.where(valid, w, lane)
                oo = pl.multiple_of(i * 16, 16)
                for f in range(5):
                    v = plsc.load_gather(fbuf, [wsafe + f * NPTS])
                    obuf[pl.ds(oo + f * _BCH, 16)] = jnp.where(valid, v, 0.0)
                return _

            lax.fori_loop(0, _BCH // 16, step, None)
            for f in range(5):
                dst = pl.multiple_of((b * 5 + f) * NCELL + cells0 + k * _BCH, 8)
                pltpu.sync_copy(
                    obuf.at[pl.ds(pl.multiple_of(f * _BCH, 8), _BCH)],
                    out_ref.at[pl.ds(dst, _BCH)])

    pl.run_scoped(phase_b,
                  pltpu.VMEM((5 * NPTS,), jnp.float32),
                  pltpu.VMEM((NCELL // 8,), jnp.int32),
                  pltpu.VMEM((5 * _BCH,), jnp.float32))


def _voxelize(radar):
    pts_t = radar.transpose(0, 2, 1).reshape(NB * 5 * NPTS)
    neg1 = jnp.full((NWIN,), -1, jnp.int32)
    mesh = plsc.VectorSubcoreMesh(core_axis_name="c", subcore_axis_name="s")
    grid = pl.kernel(
        _vox_body,
        mesh=mesh,
        out_type=jax.ShapeDtypeStruct((NB * 5 * NCELL,), jnp.float32),
        scratch_types=[pltpu.VMEM_SHARED((2 * NWIN,), jnp.int32)],
        compiler_params=pltpu.CompilerParams(needs_layout_passes=False),
    )(pts_t, neg1)
    return grid.reshape(NB, 5, YS, XS)


def _shift_w(x, dx):
    c, r = x.shape[0], x.shape[1]
    zcol = jnp.zeros((c, r, 1), jnp.float32)
    if dx == 0:
        return jnp.concatenate([zcol, x[:, :, :XS - 1]], axis=2)
    if dx == 1:
        return x
    return jnp.concatenate([x[:, :, 1:], zcol], axis=2)


_CK = 32


def _conv_body(x_ref, stats_ref, w_ref, b_ref, g_ref, be_ref,
               yc_ref, ostats_ref, xp_scr, stage_scr, sem, *,
               cin, cout, norm_in, rows):
    bi = pl.program_id(0)
    t = pl.program_id(1)

    @pl.when(t == 0)
    def _prologue():
        if norm_in:
            s1 = stats_ref[0, 0][:, None, None]
            s2 = stats_ref[0, 1][:, None, None]
            m = s1 / NCELL
            v = s2 / NCELL - m * m
            a = g_ref[...][:, :, None] * jax.lax.rsqrt(v + BN_EPS)
            c = be_ref[...][:, :, None] - m * a

        def _copy(i, buf):
            return pltpu.make_async_copy(
                x_ref.at[bi, :, pl.ds(i * _CK, _CK), :],
                stage_scr.at[buf], sem.at[buf])

        nck = YS // _CK
        _copy(0, 0).start()
        _copy(1, 1).start()
        for i in range(nck):
            _copy(i, i % 2).wait()
            xc = stage_scr[i % 2]
            if norm_in:
                xc = jnp.maximum(a * xc + c, 0.0)
            xp_scr[:, 8 + i * _CK:8 + (i + 1) * _CK, :] = xc
            if i + 2 < nck:
                _copy(i + 2, i % 2).start()
        xp_scr[:, 0:8, :] = jnp.zeros((cin, 8, XS), jnp.float32)
        xp_scr[:, YS + 8:YS + 16, :] = jnp.zeros((cin, 8, XS), jnp.float32)

    xt = xp_scr[:, pl.ds(t * rows, rows + 16), :]
    xcat = jnp.concatenate(
        [xt[:, 7 + dy:7 + dy + rows, :] for dy in range(3)], axis=0)
    xs = xcat.reshape(3 * cin, rows * XS)
    acc = jnp.zeros((cout, rows, XS), jnp.float32)
    for dx in range(3):
        p = jax.lax.dot_general(
            w_ref[dx], xs, (((1,), (0,)), ((), ())),
            preferred_element_type=jnp.float32).reshape(cout, rows, XS)
        acc = acc + _shift_w(p, dx)
    acc = acc.reshape(cout, rows * XS) + b_ref[...]
    yc_ref[...] = acc.reshape(1, cout, rows, XS)

    @pl.when(t == 0)
    def _init_stats():
        ostats_ref[...] = jnp.zeros((1, 2, cout), jnp.float32)

    ones = jnp.ones((rows * XS, 8), jnp.float32)
    ssum = jax.lax.dot_general(acc, ones, (((1,), (0,)), ((), ())),
                               preferred_element_type=jnp.float32)[:, 0]
    psq = jax.lax.dot_general(acc, acc, (((1,), (1,)), ((), ())),
                              preferred_element_type=jnp.float32)
    eye = (jax.lax.broadcasted_iota(jnp.int32, (cout, cout), 0) ==
           jax.lax.broadcasted_iota(jnp.int32, (cout, cout), 1))
    ssq = jnp.sum(jnp.where(eye, psq, 0.0), axis=1)
    ostats_ref[0, 0] = ostats_ref[0, 0] + ssum
    ostats_ref[0, 1] = ostats_ref[0, 1] + ssq


def _conv_layer(x, stats, w3, b, g, be, norm_in, rows=16):
    bsz, cin = x.shape[0], x.shape[1]
    cout = w3.shape[1]
    nt = YS // rows
    body = functools.partial(_conv_body, cin=cin, cout=cout,
                             norm_in=norm_in, rows=rows)
    return pl.pallas_call(
        body,
        grid=(bsz, nt),
        in_specs=[
            pl.BlockSpec(memory_space=pl.ANY),
            pl.BlockSpec((1, 2, cin), lambda i, t: (i, 0, 0)),
            pl.BlockSpec((3, cout, 3 * cin), lambda i, t: (0, 0, 0)),
            pl.BlockSpec((cout, 1), lambda i, t: (0, 0)),
            pl.BlockSpec((cin, 1), lambda i, t: (0, 0)),
            pl.BlockSpec((cin, 1), lambda i, t: (0, 0)),
        ],
        out_specs=[
            pl.BlockSpec((1, cout, rows, XS), lambda i, t: (i, 0, t, 0)),
            pl.BlockSpec((1, 2, cout), lambda i, t: (i, 0, 0)),
        ],
        out_shape=[
            jax.ShapeDtypeStruct((bsz, cout, YS, XS), jnp.float32),
            jax.ShapeDtypeStruct((bsz, 2, cout), jnp.float32),
        ],
        scratch_shapes=[
            pltpu.VMEM((cin, YS + 16, XS), jnp.float32),
            pltpu.VMEM((2, cin, _CK, XS), jnp.float32),
            pltpu.SemaphoreType.DMA((2,)),
        ],
    )(x, stats, w3, b, g, be)


def _final_body(y_ref, stats_ref, g_ref, be_ref, o_ref):
    s1 = stats_ref[0, 0][:, None, None]
    s2 = stats_ref[0, 1][:, None, None]
    m = s1 / NCELL
    v = s2 / NCELL - m * m
    a = g_ref[...][:, :, None] * jax.lax.rsqrt(v + BN_EPS)
    c = be_ref[...][:, :, None] - m * a
    o_ref[0] = jnp.maximum(a * y_ref[0] + c, 0.0)


def _final_norm(y, stats, g, be, rows=64):
    bsz, cout = y.shape[0], y.shape[1]
    return pl.pallas_call(
        _final_body,
        grid=(bsz, YS // rows),
        in_specs=[
            pl.BlockSpec((1, cout, rows, XS), lambda i, t: (i, 0, t, 0)),
            pl.BlockSpec((1, 2, cout), lambda i, t: (i, 0, 0)),
            pl.BlockSpec((cout, 1), lambda i, t: (0, 0)),
            pl.BlockSpec((cout, 1), lambda i, t: (0, 0)),
        ],
        out_specs=pl.BlockSpec((1, cout, rows, XS), lambda i, t: (i, 0, t, 0)),
        out_shape=jax.ShapeDtypeStruct((bsz, cout, YS, XS), jnp.float32),
    )(y, stats, g, be)


def _w3(W):
    return W.transpose(3, 0, 2, 1).reshape(3, W.shape[0], 3 * W.shape[1])


def kernel(radar_points_list, W1, b1, g1, be1, W2, b2, g2, be2,
           W3, b3, g3, be3, W4, b4, g4, be4):
    grid = _voxelize(radar_points_list)
    bsz = grid.shape[0]
    dummy_stats = jnp.zeros((bsz, 2, 5), jnp.float32)
    dummy_gb = jnp.zeros((5, 1), jnp.float32)
    h, s = _conv_layer(grid, dummy_stats, _w3(W1), b1[:, None],
                       dummy_gb, dummy_gb, norm_in=False)
    for (W, b, g, be, gp, bep) in ((W2, b2, g2, be2, g1, be1),
                                   (W3, b3, g3, be3, g2, be2),
                                   (W4, b4, g4, be4, g3, be3)):
        h, s_next = _conv_layer(h, s, _w3(W), b[:, None],
                                gp[:, None], bep[:, None], norm_in=True)
        s = s_next
    return _final_norm(h, s, g4[:, None], be4[:, None])


# speedup vs baseline: 1.0220x; 1.0117x over previous
import functools

import jax
import jax.numpy as jnp
from jax import lax
from jax.experimental import pallas as pl
from jax.experimental.pallas import tpu as pltpu
from jax.experimental.pallas import tpu_sc as plsc

X_MIN, Y_MIN, Z_MIN = -51.2, -51.2, -5.0
X_MAX, Y_MAX, Z_MAX = 51.2, 51.2, 3.0
VX, VY = 0.4, 0.4
XS = int((X_MAX - X_MIN) / VX)
YS = int((Y_MAX - Y_MIN) / VY)
NCELL = YS * XS
BN_EPS = 1e-5


NB = 4
NPTS = 20000
NWIN = NCELL + 16
_ACH = 2000
_BCH = 1024


def _vox_body(pts_ref, neg1_ref, out_ref, shared_win):
    c = lax.axis_index("c")
    s = lax.axis_index("s")
    lane = lax.iota(jnp.int32, 16)
    row = s // 8
    b = c * 2 + row
    pbase = b * 5 * NPTS

    def phase_a(win, xb, yb, zb):
        pltpu.sync_copy(neg1_ref, win)
        for ci in range(NPTS // _ACH):
            for buf, f in ((xb, 0), (yb, 1), (zb, 2)):
                off = pl.multiple_of(pbase + f * NPTS + ci * _ACH, 8)
                pltpu.sync_copy(pts_ref.at[pl.ds(off, _ACH)], buf)

            def step(i, _):
                o = pl.multiple_of(i * 16, 16)
                xv = xb[pl.ds(o, 16)]
                yv = yb[pl.ds(o, 16)]
                zv = zb[pl.ds(o, 16)]
                valid = ((xv >= X_MIN) & (xv < X_MAX) &
                         (yv >= Y_MIN) & (yv < Y_MAX) &
                         (zv >= Z_MIN) & (zv < Z_MAX))
                xi = jnp.clip(((xv - X_MIN) / VX).astype(jnp.int32), 0, XS - 1)
                yi = jnp.clip(((yv - Y_MIN) / VY).astype(jnp.int32), 0, YS - 1)
                vox = jnp.where(valid, yi * XS + xi, NCELL + lane)
                pidx = ci * _ACH + i * 16 + lane
                plsc.store_scatter(win, [vox], pidx)
                return _

            lax.fori_loop(0, _ACH // 16, step, None)
        woff = pl.multiple_of(row * NWIN, 8)
        pltpu.sync_copy(win, shared_win.at[pl.ds(woff, NWIN)])

    @pl.when((s == 0) | (s == 8))
    def _():
        pl.run_scoped(phase_a,
                      pltpu.VMEM((NWIN,), jnp.int32),
                      pltpu.VMEM((_ACH,), jnp.float32),
                      pltpu.VMEM((_ACH,), jnp.float32),
                      pltpu.VMEM((_ACH,), jnp.float32))

    plsc.subcore_barrier()

    sl = s % 8
    cells0 = sl * (NCELL // 8)

    def phase_b(fbuf, wsl, obuf):
        pltpu.sync_copy(pts_ref.at[pl.ds(pl.multiple_of(pbase, 8), 5 * NPTS)],
                        fbuf)
        pltpu.sync_copy(
            shared_win.at[pl.ds(pl.multiple_of(row * NWIN + cells0, 8),
                                NCELL // 8)], wsl)
        for k in range(NCELL // 8 // _BCH):

            def step(i, _):
                o = pl.multiple_of(k * _BCH + i * 16, 16)
                w = wsl[pl.ds(o, 16)]
                valid = w >= 0
                wsafe = jnp.where(valid, w, lane)
                oo = pl.multiple_of(i * 16, 16)
                for f in range(5):
                    v = plsc.load_gather(fbuf, [wsafe + f * NPTS])
                    obuf[pl.ds(oo + f * _BCH, 16)] = jnp.where(valid, v, 0.0)
                return _

            lax.fori_loop(0, _BCH // 16, step, None)
            for f in range(5):
                dst = pl.multiple_of((b * 5 + f) * NCELL + cells0 + k * _BCH, 8)
                pltpu.sync_copy(
                    obuf.at[pl.ds(pl.multiple_of(f * _BCH, 8), _BCH)],
                    out_ref.at[pl.ds(dst, _BCH)])

    pl.run_scoped(phase_b,
                  pltpu.VMEM((5 * NPTS,), jnp.float32),
                  pltpu.VMEM((NCELL // 8,), jnp.int32),
                  pltpu.VMEM((5 * _BCH,), jnp.float32))


def _voxelize(radar):
    pts_t = radar.transpose(0, 2, 1).reshape(NB * 5 * NPTS)
    neg1 = jnp.full((NWIN,), -1, jnp.int32)
    mesh = plsc.VectorSubcoreMesh(core_axis_name="c", subcore_axis_name="s")
    grid = pl.kernel(
        _vox_body,
        mesh=mesh,
        out_type=jax.ShapeDtypeStruct((NB * 5 * NCELL,), jnp.float32),
        scratch_types=[pltpu.VMEM_SHARED((2 * NWIN,), jnp.int32)],
        compiler_params=pltpu.CompilerParams(needs_layout_passes=False),
    )(pts_t, neg1)
    return grid.reshape(NB, 5, YS, XS)


def _shift_w(x, dx):
    c, r = x.shape[0], x.shape[1]
    zcol = jnp.zeros((c, r, 1), jnp.float32)
    if dx == 0:
        return jnp.concatenate([zcol, x[:, :, :XS - 1]], axis=2)
    if dx == 1:
        return x
    return jnp.concatenate([x[:, :, 1:], zcol], axis=2)


_CK = 32


def _conv_body(x_ref, stats_ref, w_ref, b_ref, g_ref, be_ref,
               yc_ref, ostats_ref, xp_scr, stage_scr, sem, *,
               cin, cout, norm_in, rows):
    bi = pl.program_id(0)
    t = pl.program_id(1)

    @pl.when(t == 0)
    def _prologue():
        if norm_in:
            s1 = stats_ref[0, 0][:, None, None]
            s2 = stats_ref[0, 1][:, None, None]
            m = s1 / NCELL
            v = s2 / NCELL - m * m
            a = g_ref[...][:, :, None] * jax.lax.rsqrt(v + BN_EPS)
            c = be_ref[...][:, :, None] - m * a

        def _copy(i, buf):
            return pltpu.make_async_copy(
                x_ref.at[bi, :, pl.ds(i * _CK, _CK), :],
                stage_scr.at[buf], sem.at[buf])

        nck = YS // _CK
        _copy(0, 0).start()
        _copy(1, 1).start()
        for i in range(nck):
            _copy(i, i % 2).wait()
            xc = stage_scr[i % 2]
            if norm_in:
                xc = jnp.maximum(a * xc + c, 0.0)
            xp_scr[:, 8 + i * _CK:8 + (i + 1) * _CK, :] = xc.astype(jnp.bfloat16)
            if i + 2 < nck:
                _copy(i + 2, i % 2).start()
        xp_scr[:, 0:8, :] = jnp.zeros((cin, 8, XS), jnp.bfloat16)
        xp_scr[:, YS + 8:YS + 16, :] = jnp.zeros((cin, 8, XS), jnp.bfloat16)

    xt = xp_scr[:, pl.ds(t * rows, rows + 16), :]
    xcat = jnp.concatenate(
        [xt[:, 7 + dy:7 + dy + rows, :] for dy in range(3)], axis=0)
    xs = xcat.reshape(3 * cin, rows * XS)
    acc = jnp.zeros((cout, rows, XS), jnp.float32)
    for dx in range(3):
        p = jax.lax.dot_general(
            w_ref[dx], xs, (((1,), (0,)), ((), ())),
            preferred_element_type=jnp.float32).reshape(cout, rows, XS)
        acc = acc + _shift_w(p, dx)
    acc = acc.reshape(cout, rows * XS) + b_ref[...]
    yc_ref[...] = acc.reshape(1, cout, rows, XS)

    @pl.when(t == 0)
    def _init_stats():
        ostats_ref[...] = jnp.zeros((1, 2, cout), jnp.float32)

    ones = jnp.ones((rows * XS, 8), jnp.float32)
    ssum = jax.lax.dot_general(acc, ones, (((1,), (0,)), ((), ())),
                               preferred_element_type=jnp.float32)[:, 0]
    psq = jax.lax.dot_general(acc, acc, (((1,), (1,)), ((), ())),
                              preferred_element_type=jnp.float32)
    eye = (jax.lax.broadcasted_iota(jnp.int32, (cout, cout), 0) ==
           jax.lax.broadcasted_iota(jnp.int32, (cout, cout), 1))
    ssq = jnp.sum(jnp.where(eye, psq, 0.0), axis=1)
    ostats_ref[0, 0] = ostats_ref[0, 0] + ssum
    ostats_ref[0, 1] = ostats_ref[0, 1] + ssq


def _conv_layer(x, stats, w3, b, g, be, norm_in, rows=16):
    bsz, cin = x.shape[0], x.shape[1]
    cout = w3.shape[1]
    nt = YS // rows
    body = functools.partial(_conv_body, cin=cin, cout=cout,
                             norm_in=norm_in, rows=rows)
    return pl.pallas_call(
        body,
        grid=(bsz, nt),
        in_specs=[
            pl.BlockSpec(memory_space=pl.ANY),
            pl.BlockSpec((1, 2, cin), lambda i, t: (i, 0, 0)),
            pl.BlockSpec((3, cout, 3 * cin), lambda i, t: (0, 0, 0)),
            pl.BlockSpec((cout, 1), lambda i, t: (0, 0)),
            pl.BlockSpec((cin, 1), lambda i, t: (0, 0)),
            pl.BlockSpec((cin, 1), lambda i, t: (0, 0)),
        ],
        out_specs=[
            pl.BlockSpec((1, cout, rows, XS), lambda i, t: (i, 0, t, 0)),
            pl.BlockSpec((1, 2, cout), lambda i, t: (i, 0, 0)),
        ],
        out_shape=[
            jax.ShapeDtypeStruct((bsz, cout, YS, XS), jnp.float32),
            jax.ShapeDtypeStruct((bsz, 2, cout), jnp.float32),
        ],
        scratch_shapes=[
            pltpu.VMEM((cin, YS + 16, XS), jnp.bfloat16),
            pltpu.VMEM((2, cin, _CK, XS), jnp.float32),
            pltpu.SemaphoreType.DMA((2,)),
        ],
    )(x, stats, w3, b, g, be)


def _final_body(y_ref, stats_ref, g_ref, be_ref, o_ref):
    s1 = stats_ref[0, 0][:, None, None]
    s2 = stats_ref[0, 1][:, None, None]
    m = s1 / NCELL
    v = s2 / NCELL - m * m
    a = g_ref[...][:, :, None] * jax.lax.rsqrt(v + BN_EPS)
    c = be_ref[...][:, :, None] - m * a
    o_ref[0] = jnp.maximum(a * y_ref[0] + c, 0.0)


def _final_norm(y, stats, g, be, rows=64):
    bsz, cout = y.shape[0], y.shape[1]
    return pl.pallas_call(
        _final_body,
        grid=(bsz, YS // rows),
        in_specs=[
            pl.BlockSpec((1, cout, rows, XS), lambda i, t: (i, 0, t, 0)),
            pl.BlockSpec((1, 2, cout), lambda i, t: (i, 0, 0)),
            pl.BlockSpec((cout, 1), lambda i, t: (0, 0)),
            pl.BlockSpec((cout, 1), lambda i, t: (0, 0)),
        ],
        out_specs=pl.BlockSpec((1, cout, rows, XS), lambda i, t: (i, 0, t, 0)),
        out_shape=jax.ShapeDtypeStruct((bsz, cout, YS, XS), jnp.float32),
    )(y, stats, g, be)


def _w3(W):
    return W.transpose(3, 0, 2, 1).reshape(
        3, W.shape[0], 3 * W.shape[1]).astype(jnp.bfloat16)


def kernel(radar_points_list, W1, b1, g1, be1, W2, b2, g2, be2,
           W3, b3, g3, be3, W4, b4, g4, be4):
    grid = _voxelize(radar_points_list)
    bsz = grid.shape[0]
    dummy_stats = jnp.zeros((bsz, 2, 5), jnp.float32)
    dummy_gb = jnp.zeros((5, 1), jnp.float32)
    h, s = _conv_layer(grid, dummy_stats, _w3(W1), b1[:, None],
                       dummy_gb, dummy_gb, norm_in=False)
    for (W, b, g, be, gp, bep) in ((W2, b2, g2, be2, g1, be1),
                                   (W3, b3, g3, be3, g2, be2),
                                   (W4, b4, g4, be4, g3, be3)):
        h, s_next = _conv_layer(h, s, _w3(W), b[:, None],
                                gp[:, None], bep[:, None], norm_in=True)
        s = s_next
    return _final_norm(h, s, g4[:, None], be4[:, None])


# speedup vs baseline: 1.0517x; 1.0291x over previous
import functools

import jax
import jax.numpy as jnp
from jax import lax
from jax.experimental import pallas as pl
from jax.experimental.pallas import tpu as pltpu
from jax.experimental.pallas import tpu_sc as plsc

X_MIN, Y_MIN, Z_MIN = -51.2, -51.2, -5.0
X_MAX, Y_MAX, Z_MAX = 51.2, 51.2, 3.0
VX, VY = 0.4, 0.4
XS = int((X_MAX - X_MIN) / VX)
YS = int((Y_MAX - Y_MIN) / VY)
NCELL = YS * XS
BN_EPS = 1e-5


NB = 4
NPTS = 20000
NWIN = NCELL + 16
_ACH = 2000
_BCH = 1024


def _vox_body(pts_ref, neg1_ref, out_ref, shared_win):
    c = lax.axis_index("c")
    s = lax.axis_index("s")
    lane = lax.iota(jnp.int32, 16)
    row = s // 8
    b = c * 2 + row
    pbase = b * 5 * NPTS

    def phase_a(win, xb, yb, zb):
        pltpu.sync_copy(neg1_ref, win)
        for ci in range(NPTS // _ACH):
            for buf, f in ((xb, 0), (yb, 1), (zb, 2)):
                off = pl.multiple_of(pbase + f * NPTS + ci * _ACH, 8)
                pltpu.sync_copy(pts_ref.at[pl.ds(off, _ACH)], buf)

            def step(i, _):
                o = pl.multiple_of(i * 16, 16)
                xv = xb[pl.ds(o, 16)]
                yv = yb[pl.ds(o, 16)]
                zv = zb[pl.ds(o, 16)]
                valid = ((xv >= X_MIN) & (xv < X_MAX) &
                         (yv >= Y_MIN) & (yv < Y_MAX) &
                         (zv >= Z_MIN) & (zv < Z_MAX))
                xi = jnp.clip(((xv - X_MIN) / VX).astype(jnp.int32), 0, XS - 1)
                yi = jnp.clip(((yv - Y_MIN) / VY).astype(jnp.int32), 0, YS - 1)
                vox = jnp.where(valid, yi * XS + xi, NCELL + lane)
                pidx = ci * _ACH + i * 16 + lane
                plsc.store_scatter(win, [vox], pidx)
                return _

            lax.fori_loop(0, _ACH // 16, step, None)
        woff = pl.multiple_of(row * NWIN, 8)
        pltpu.sync_copy(win, shared_win.at[pl.ds(woff, NWIN)])

    @pl.when((s == 0) | (s == 8))
    def _():
        pl.run_scoped(phase_a,
                      pltpu.VMEM((NWIN,), jnp.int32),
                      pltpu.VMEM((_ACH,), jnp.float32),
                      pltpu.VMEM((_ACH,), jnp.float32),
                      pltpu.VMEM((_ACH,), jnp.float32))

    plsc.subcore_barrier()

    sl = s % 8
    cells0 = sl * (NCELL // 8)

    def phase_b(fbuf, wsl, obuf):
        pltpu.sync_copy(pts_ref.at[pl.ds(pl.multiple_of(pbase, 8), 5 * NPTS)],
                        fbuf)
        pltpu.sync_copy(
            shared_win.at[pl.ds(pl.multiple_of(row * NWIN + cells0, 8),
                                NCELL // 8)], wsl)
        for k in range(NCELL // 8 // _BCH):

            def step(i, _):
                o = pl.multiple_of(k * _BCH + i * 16, 16)
                w = wsl[pl.ds(o, 16)]
                valid = w >= 0
                wsafe = jnp.where(valid, w, lane)
                oo = pl.multiple_of(i * 16, 16)
                for f in range(5):
                    v = plsc.load_gather(fbuf, [wsafe + f * NPTS])
                    obuf[pl.ds(oo + f * _BCH, 16)] = jnp.where(valid, v, 0.0)
                return _

            lax.fori_loop(0, _BCH // 16, step, None)
            for f in range(5):
                dst = pl.multiple_of((b * 5 + f) * NCELL + cells0 + k * _BCH, 8)
                pltpu.sync_copy(
                    obuf.at[pl.ds(pl.multiple_of(f * _BCH, 8), _BCH)],
                    out_ref.at[pl.ds(dst, _BCH)])

    pl.run_scoped(phase_b,
                  pltpu.VMEM((5 * NPTS,), jnp.float32),
                  pltpu.VMEM((NCELL // 8,), jnp.int32),
                  pltpu.VMEM((5 * _BCH,), jnp.float32))


def _voxelize(radar):
    pts_t = radar.transpose(0, 2, 1).reshape(NB * 5 * NPTS)
    neg1 = jnp.full((NWIN,), -1, jnp.int32)
    mesh = plsc.VectorSubcoreMesh(core_axis_name="c", subcore_axis_name="s")
    grid = pl.kernel(
        _vox_body,
        mesh=mesh,
        out_type=jax.ShapeDtypeStruct((NB * 5 * NCELL,), jnp.float32),
        scratch_types=[pltpu.VMEM_SHARED((2 * NWIN,), jnp.int32)],
        compiler_params=pltpu.CompilerParams(needs_layout_passes=False),
    )(pts_t, neg1)
    return grid.reshape(NB, 5, YS, XS)


def _shift_w(x, dx):
    c, r = x.shape[0], x.shape[1]
    zcol = jnp.zeros((c, r, 1), jnp.float32)
    if dx == 0:
        return jnp.concatenate([zcol, x[:, :, :XS - 1]], axis=2)
    if dx == 1:
        return x
    return jnp.concatenate([x[:, :, 1:], zcol], axis=2)


_CK = 32


def _conv_body(x_ref, stats_ref, w_ref, b_ref, g_ref, be_ref,
               yc_ref, ostats_ref, xp_scr, stage_scr, sem, *,
               cin, cout, norm_in, rows):
    bi = pl.program_id(0)
    t = pl.program_id(1)

    @pl.when(t == 0)
    def _prologue():
        if norm_in:
            s1 = stats_ref[0, 0][:, None, None]
            s2 = stats_ref[0, 1][:, None, None]
            m = s1 / NCELL
            v = s2 / NCELL - m * m
            a = g_ref[...][:, :, None] * jax.lax.rsqrt(v + BN_EPS)
            c = be_ref[...][:, :, None] - m * a

        def _copy(i, buf):
            return pltpu.make_async_copy(
                x_ref.at[bi, :, pl.ds(i * _CK, _CK), :],
                stage_scr.at[buf], sem.at[buf])

        nck = YS // _CK
        _copy(0, 0).start()
        _copy(1, 1).start()
        for i in range(nck):
            _copy(i, i % 2).wait()
            xc = stage_scr[i % 2]
            if norm_in:
                xc = jnp.maximum(a * xc.astype(jnp.float32) + c, 0.0)
            xp_scr[:, 8 + i * _CK:8 + (i + 1) * _CK, :] = xc.astype(jnp.bfloat16)
            if i + 2 < nck:
                _copy(i + 2, i % 2).start()
        xp_scr[:, 0:8, :] = jnp.zeros((cin, 8, XS), jnp.bfloat16)
        xp_scr[:, YS + 8:YS + 16, :] = jnp.zeros((cin, 8, XS), jnp.bfloat16)

    xt = xp_scr[:, pl.ds(t * rows, rows + 16), :]
    xcat = jnp.concatenate(
        [xt[:, 7 + dy:7 + dy + rows, :] for dy in range(3)], axis=0)
    xs = xcat.reshape(3 * cin, rows * XS)
    acc = jnp.zeros((cout, rows, XS), jnp.float32)
    for dx in range(3):
        p = jax.lax.dot_general(
            w_ref[dx], xs, (((1,), (0,)), ((), ())),
            preferred_element_type=jnp.float32).reshape(cout, rows, XS)
        acc = acc + _shift_w(p, dx)
    acc = acc.reshape(cout, rows * XS) + b_ref[...]
    yc_ref[...] = acc.reshape(1, cout, rows, XS).astype(jnp.bfloat16)

    @pl.when(t == 0)
    def _init_stats():
        ostats_ref[...] = jnp.zeros((1, 2, cout), jnp.float32)

    ones = jnp.ones((rows * XS, 8), jnp.float32)
    ssum = jax.lax.dot_general(acc, ones, (((1,), (0,)), ((), ())),
                               preferred_element_type=jnp.float32)[:, 0]
    psq = jax.lax.dot_general(acc, acc, (((1,), (1,)), ((), ())),
                              preferred_element_type=jnp.float32)
    eye = (jax.lax.broadcasted_iota(jnp.int32, (cout, cout), 0) ==
           jax.lax.broadcasted_iota(jnp.int32, (cout, cout), 1))
    ssq = jnp.sum(jnp.where(eye, psq, 0.0), axis=1)
    ostats_ref[0, 0] = ostats_ref[0, 0] + ssum
    ostats_ref[0, 1] = ostats_ref[0, 1] + ssq


def _conv_layer(x, stats, w3, b, g, be, norm_in, rows=16):
    bsz, cin = x.shape[0], x.shape[1]
    cout = w3.shape[1]
    nt = YS // rows
    body = functools.partial(_conv_body, cin=cin, cout=cout,
                             norm_in=norm_in, rows=rows)
    return pl.pallas_call(
        body,
        grid=(bsz, nt),
        in_specs=[
            pl.BlockSpec(memory_space=pl.ANY),
            pl.BlockSpec((1, 2, cin), lambda i, t: (i, 0, 0)),
            pl.BlockSpec((3, cout, 3 * cin), lambda i, t: (0, 0, 0)),
            pl.BlockSpec((cout, 1), lambda i, t: (0, 0)),
            pl.BlockSpec((cin, 1), lambda i, t: (0, 0)),
            pl.BlockSpec((cin, 1), lambda i, t: (0, 0)),
        ],
        out_specs=[
            pl.BlockSpec((1, cout, rows, XS), lambda i, t: (i, 0, t, 0)),
            pl.BlockSpec((1, 2, cout), lambda i, t: (i, 0, 0)),
        ],
        out_shape=[
            jax.ShapeDtypeStruct((bsz, cout, YS, XS), jnp.bfloat16),
            jax.ShapeDtypeStruct((bsz, 2, cout), jnp.float32),
        ],
        scratch_shapes=[
            pltpu.VMEM((cin, YS + 16, XS), jnp.bfloat16),
            pltpu.VMEM((2, cin, _CK, XS), x.dtype),
            pltpu.SemaphoreType.DMA((2,)),
        ],
    )(x, stats, w3, b, g, be)


def _final_body(y_ref, stats_ref, g_ref, be_ref, o_ref):
    s1 = stats_ref[0, 0][:, None, None]
    s2 = stats_ref[0, 1][:, None, None]
    m = s1 / NCELL
    v = s2 / NCELL - m * m
    a = g_ref[...][:, :, None] * jax.lax.rsqrt(v + BN_EPS)
    c = be_ref[...][:, :, None] - m * a
    o_ref[0] = jnp.maximum(a * y_ref[0].astype(jnp.float32) + c, 0.0)


def _final_norm(y, stats, g, be, rows=64):
    bsz, cout = y.shape[0], y.shape[1]
    return pl.pallas_call(
        _final_body,
        grid=(bsz, YS // rows),
        in_specs=[
            pl.BlockSpec((1, cout, rows, XS), lambda i, t: (i, 0, t, 0)),
            pl.BlockSpec((1, 2, cout), lambda i, t: (i, 0, 0)),
            pl.BlockSpec((cout, 1), lambda i, t: (0, 0)),
            pl.BlockSpec((cout, 1), lambda i, t: (0, 0)),
        ],
        out_specs=pl.BlockSpec((1, cout, rows, XS), lambda i, t: (i, 0, t, 0)),
        out_shape=jax.ShapeDtypeStruct((bsz, cout, YS, XS), jnp.float32),
    )(y, stats, g, be)


def _w3(W):
    return W.transpose(3, 0, 2, 1).reshape(
        3, W.shape[0], 3 * W.shape[1]).astype(jnp.bfloat16)


def kernel(radar_points_list, W1, b1, g1, be1, W2, b2, g2, be2,
           W3, b3, g3, be3, W4, b4, g4, be4):
    grid = _voxelize(radar_points_list)
    bsz = grid.shape[0]
    dummy_stats = jnp.zeros((bsz, 2, 5), jnp.float32)
    dummy_gb = jnp.zeros((5, 1), jnp.float32)
    h, s = _conv_layer(grid, dummy_stats, _w3(W1), b1[:, None],
                       dummy_gb, dummy_gb, norm_in=False)
    for (W, b, g, be, gp, bep) in ((W2, b2, g2, be2, g1, be1),
                                   (W3, b3, g3, be3, g2, be2),
                                   (W4, b4, g4, be4, g3, be3)):
        h, s_next = _conv_layer(h, s, _w3(W), b[:, None],
                                gp[:, None], bep[:, None], norm_in=True)
        s = s_next
    return _final_norm(h, s, g4[:, None], be4[:, None])


# speedup vs baseline: 1.1185x; 1.0635x over previous
import functools

import jax
import jax.numpy as jnp
from jax import lax
from jax.experimental import pallas as pl
from jax.experimental.pallas import tpu as pltpu
from jax.experimental.pallas import tpu_sc as plsc

X_MIN, Y_MIN, Z_MIN = -51.2, -51.2, -5.0
X_MAX, Y_MAX, Z_MAX = 51.2, 51.2, 3.0
VX, VY = 0.4, 0.4
XS = int((X_MAX - X_MIN) / VX)
YS = int((Y_MAX - Y_MIN) / VY)
NCELL = YS * XS
BN_EPS = 1e-5


NB = 4
NPTS = 20000
NWIN = NCELL + 16
_ACH = 2000
_BCH = 1024


def _vox_body(pts_ref, neg1_ref, out_ref, shared_win):
    c = lax.axis_index("c")
    s = lax.axis_index("s")
    lane = lax.iota(jnp.int32, 16)
    row = s // 8
    b = c * 2 + row
    pbase = b * 5 * NPTS

    def phase_a(win, xb, yb, zb):
        pltpu.sync_copy(neg1_ref, win)
        for ci in range(NPTS // _ACH):
            for buf, f in ((xb, 0), (yb, 1), (zb, 2)):
                off = pl.multiple_of(pbase + f * NPTS + ci * _ACH, 8)
                pltpu.sync_copy(pts_ref.at[pl.ds(off, _ACH)], buf)

            def step(i, _):
                o = pl.multiple_of(i * 16, 16)
                xv = xb[pl.ds(o, 16)]
                yv = yb[pl.ds(o, 16)]
                zv = zb[pl.ds(o, 16)]
                valid = ((xv >= X_MIN) & (xv < X_MAX) &
                         (yv >= Y_MIN) & (yv < Y_MAX) &
                         (zv >= Z_MIN) & (zv < Z_MAX))
                xi = jnp.clip(((xv - X_MIN) / VX).astype(jnp.int32), 0, XS - 1)
                yi = jnp.clip(((yv - Y_MIN) / VY).astype(jnp.int32), 0, YS - 1)
                vox = jnp.where(valid, yi * XS + xi, NCELL + lane)
                pidx = ci * _ACH + i * 16 + lane
                plsc.store_scatter(win, [vox], pidx)
                return _

            lax.fori_loop(0, _ACH // 16, step, None)
        woff = pl.multiple_of(row * NWIN, 8)
        pltpu.sync_copy(win, shared_win.at[pl.ds(woff, NWIN)])

    @pl.when((s == 0) | (s == 8))
    def _():
        pl.run_scoped(phase_a,
                      pltpu.VMEM((NWIN,), jnp.int32),
                      pltpu.VMEM((_ACH,), jnp.float32),
                      pltpu.VMEM((_ACH,), jnp.float32),
                      pltpu.VMEM((_ACH,), jnp.float32))

    plsc.subcore_barrier()

    sl = s % 8
    cells0 = sl * (NCELL // 8)

    def phase_b(fbuf, wsl, obuf):
        pltpu.sync_copy(pts_ref.at[pl.ds(pl.multiple_of(pbase, 8), 5 * NPTS)],
                        fbuf)
        pltpu.sync_copy(
            shared_win.at[pl.ds(pl.multiple_of(row * NWIN + cells0, 8),
                                NCELL // 8)], wsl)
        for k in range(NCELL // 8 // _BCH):

            def step(i, _):
                o = pl.multiple_of(k * _BCH + i * 16, 16)
                w = wsl[pl.ds(o, 16)]
                valid = w >= 0
                wsafe = jnp.where(valid, w, lane)
                oo = pl.multiple_of(i * 16, 16)
                for f in range(5):
                    v = plsc.load_gather(fbuf, [wsafe + f * NPTS])
                    obuf[pl.ds(oo + f * _BCH, 16)] = jnp.where(valid, v, 0.0)
                return _

            lax.fori_loop(0, _BCH // 16, step, None)
            for f in range(5):
                dst = pl.multiple_of((b * 5 + f) * NCELL + cells0 + k * _BCH, 8)
                pltpu.sync_copy(
                    obuf.at[pl.ds(pl.multiple_of(f * _BCH, 8), _BCH)],
                    out_ref.at[pl.ds(dst, _BCH)])

    pl.run_scoped(phase_b,
                  pltpu.VMEM((5 * NPTS,), jnp.float32),
                  pltpu.VMEM((NCELL // 8,), jnp.int32),
                  pltpu.VMEM((5 * _BCH,), jnp.float32))


def _voxelize(radar):
    pts_t = radar.transpose(0, 2, 1).reshape(NB * 5 * NPTS)
    neg1 = jnp.full((NWIN,), -1, jnp.int32)
    mesh = plsc.VectorSubcoreMesh(core_axis_name="c", subcore_axis_name="s")
    grid = pl.kernel(
        _vox_body,
        mesh=mesh,
        out_type=jax.ShapeDtypeStruct((NB * 5 * NCELL,), jnp.float32),
        scratch_types=[pltpu.VMEM_SHARED((2 * NWIN,), jnp.int32)],
        compiler_params=pltpu.CompilerParams(needs_layout_passes=False),
    )(pts_t, neg1)
    return grid.reshape(NB, 5, YS, XS)


def _shift_w(x, dx):
    c, r = x.shape[0], x.shape[1]
    zcol = jnp.zeros((c, r, 1), jnp.float32)
    if dx == 0:
        return jnp.concatenate([zcol, x[:, :, :XS - 1]], axis=2)
    if dx == 1:
        return x
    return jnp.concatenate([x[:, :, 1:], zcol], axis=2)


_CK = 32


def _conv_body(x_ref, stats_ref, w_ref, b_ref, g_ref, be_ref,
               yc_ref, ostats_ref, xp_scr, stage_scr, sem, *,
               cin, cout, norm_in, rows):
    bi = pl.program_id(0)
    t = pl.program_id(1)

    @pl.when(t == 0)
    def _prologue():
        if norm_in:
            s1 = stats_ref[0, 0][:, None, None]
            s2 = stats_ref[0, 1][:, None, None]
            m = s1 / NCELL
            v = s2 / NCELL - m * m
            a = g_ref[...][:, :, None] * jax.lax.rsqrt(v + BN_EPS)
            c = be_ref[...][:, :, None] - m * a

        def _copy(i, buf):
            return pltpu.make_async_copy(
                x_ref.at[bi, :, pl.ds(i * _CK, _CK), :],
                stage_scr.at[buf], sem.at[buf])

        nck = YS // _CK
        _copy(0, 0).start()
        _copy(1, 1).start()
        for i in range(nck):
            _copy(i, i % 2).wait()
            xc = stage_scr[i % 2]
            if norm_in:
                xc = jnp.maximum(a * xc.astype(jnp.float32) + c, 0.0)
            xp_scr[:, 8 + i * _CK:8 + (i + 1) * _CK, :] = xc.astype(jnp.bfloat16)
            if i + 2 < nck:
                _copy(i + 2, i % 2).start()
        xp_scr[:, 0:8, :] = jnp.zeros((cin, 8, XS), jnp.bfloat16)
        xp_scr[:, YS + 8:YS + 16, :] = jnp.zeros((cin, 8, XS), jnp.bfloat16)

    xt = xp_scr[:, pl.ds(t * rows, rows + 16), :]
    xcat = jnp.concatenate(
        [xt[:, 7 + dy:7 + dy + rows, :] for dy in range(3)], axis=0)
    xs = xcat.reshape(3 * cin, rows * XS)
    acc = jnp.zeros((cout, rows, XS), jnp.float32)
    for dx in range(3):
        p = jax.lax.dot_general(
            w_ref[dx], xs, (((1,), (0,)), ((), ())),
            preferred_element_type=jnp.float32).reshape(cout, rows, XS)
        acc = acc + _shift_w(p, dx)
    acc = acc.reshape(cout, rows * XS) + b_ref[...]
    yc_ref[...] = acc.reshape(1, cout, rows, XS).astype(jnp.bfloat16)

    @pl.when(t == 0)
    def _init_stats():
        ostats_ref[...] = jnp.zeros((1, 2, cout), jnp.float32)

    ones = jnp.ones((rows * XS, 8), jnp.float32)
    ssum = jax.lax.dot_general(acc, ones, (((1,), (0,)), ((), ())),
                               preferred_element_type=jnp.float32)[:, 0]
    psq = jax.lax.dot_general(acc, acc, (((1,), (1,)), ((), ())),
                              preferred_element_type=jnp.float32)
    eye = (jax.lax.broadcasted_iota(jnp.int32, (cout, cout), 0) ==
           jax.lax.broadcasted_iota(jnp.int32, (cout, cout), 1))
    ssq = jnp.sum(jnp.where(eye, psq, 0.0), axis=1)
    ostats_ref[0, 0] = ostats_ref[0, 0] + ssum
    ostats_ref[0, 1] = ostats_ref[0, 1] + ssq


def _conv_layer(x, stats, w3, b, g, be, norm_in, rows=32):
    bsz, cin = x.shape[0], x.shape[1]
    cout = w3.shape[1]
    nt = YS // rows
    body = functools.partial(_conv_body, cin=cin, cout=cout,
                             norm_in=norm_in, rows=rows)
    return pl.pallas_call(
        body,
        grid=(bsz, nt),
        in_specs=[
            pl.BlockSpec(memory_space=pl.ANY),
            pl.BlockSpec((1, 2, cin), lambda i, t: (i, 0, 0)),
            pl.BlockSpec((3, cout, 3 * cin), lambda i, t: (0, 0, 0)),
            pl.BlockSpec((cout, 1), lambda i, t: (0, 0)),
            pl.BlockSpec((cin, 1), lambda i, t: (0, 0)),
            pl.BlockSpec((cin, 1), lambda i, t: (0, 0)),
        ],
        out_specs=[
            pl.BlockSpec((1, cout, rows, XS), lambda i, t: (i, 0, t, 0)),
            pl.BlockSpec((1, 2, cout), lambda i, t: (i, 0, 0)),
        ],
        out_shape=[
            jax.ShapeDtypeStruct((bsz, cout, YS, XS), jnp.bfloat16),
            jax.ShapeDtypeStruct((bsz, 2, cout), jnp.float32),
        ],
        scratch_shapes=[
            pltpu.VMEM((cin, YS + 16, XS), jnp.bfloat16),
            pltpu.VMEM((2, cin, _CK, XS), x.dtype),
            pltpu.SemaphoreType.DMA((2,)),
        ],
    )(x, stats, w3, b, g, be)


def _final_body(y_ref, stats_ref, g_ref, be_ref, o_ref):
    s1 = stats_ref[0, 0][:, None, None]
    s2 = stats_ref[0, 1][:, None, None]
    m = s1 / NCELL
    v = s2 / NCELL - m * m
    a = g_ref[...][:, :, None] * jax.lax.rsqrt(v + BN_EPS)
    c = be_ref[...][:, :, None] - m * a
    o_ref[0] = jnp.maximum(a * y_ref[0].astype(jnp.float32) + c, 0.0)


def _final_norm(y, stats, g, be, rows=64):
    bsz, cout = y.shape[0], y.shape[1]
    return pl.pallas_call(
        _final_body,
        grid=(bsz, YS // rows),
        in_specs=[
            pl.BlockSpec((1, cout, rows, XS), lambda i, t: (i, 0, t, 0)),
            pl.BlockSpec((1, 2, cout), lambda i, t: (i, 0, 0)),
            pl.BlockSpec((cout, 1), lambda i, t: (0, 0)),
            pl.BlockSpec((cout, 1), lambda i, t: (0, 0)),
        ],
        out_specs=pl.BlockSpec((1, cout, rows, XS), lambda i, t: (i, 0, t, 0)),
        out_shape=jax.ShapeDtypeStruct((bsz, cout, YS, XS), jnp.float32),
    )(y, stats, g, be)


def _w3(W):
    return W.transpose(3, 0, 2, 1).reshape(
        3, W.shape[0], 3 * W.shape[1]).astype(jnp.bfloat16)


def kernel(radar_points_list, W1, b1, g1, be1, W2, b2, g2, be2,
           W3, b3, g3, be3, W4, b4, g4, be4):
    grid = _voxelize(radar_points_list)
    bsz = grid.shape[0]
    dummy_stats = jnp.zeros((bsz, 2, 5), jnp.float32)
    dummy_gb = jnp.zeros((5, 1), jnp.float32)
    h, s = _conv_layer(grid, dummy_stats, _w3(W1), b1[:, None],
                       dummy_gb, dummy_gb, norm_in=False)
    for (W, b, g, be, gp, bep) in ((W2, b2, g2, be2, g1, be1),
                                   (W3, b3, g3, be3, g2, be2),
                                   (W4, b4, g4, be4, g3, be3)):
        h, s_next = _conv_layer(h, s, _w3(W), b[:, None],
                                gp[:, None], bep[:, None], norm_in=True)
        s = s_next
    return _final_norm(h, s, g4[:, None], be4[:, None])


# speedup vs baseline: 1.1327x; 1.0127x over previous
import functools

import jax
import jax.numpy as jnp
from jax import lax
from jax.experimental import pallas as pl
from jax.experimental.pallas import tpu as pltpu
from jax.experimental.pallas import tpu_sc as plsc

X_MIN, Y_MIN, Z_MIN = -51.2, -51.2, -5.0
X_MAX, Y_MAX, Z_MAX = 51.2, 51.2, 3.0
VX, VY = 0.4, 0.4
XS = int((X_MAX - X_MIN) / VX)
YS = int((Y_MAX - Y_MIN) / VY)
NCELL = YS * XS
BN_EPS = 1e-5


NB = 4
NPTS = 20000
NWIN = NCELL + 16
_ACH = 2000
_BCH = 1024


def _vox_body(pts_ref, neg1_ref, out_ref, shared_win):
    c = lax.axis_index("c")
    s = lax.axis_index("s")
    lane = lax.iota(jnp.int32, 16)
    row = s // 8
    b = c * 2 + row
    pbase = b * 5 * NPTS

    def phase_a(win, xb, yb, zb):
        pltpu.sync_copy(neg1_ref, win)
        for ci in range(NPTS // _ACH):
            for buf, f in ((xb, 0), (yb, 1), (zb, 2)):
                off = pl.multiple_of(pbase + f * NPTS + ci * _ACH, 8)
                pltpu.sync_copy(pts_ref.at[pl.ds(off, _ACH)], buf)

            def step(i, _):
                o = pl.multiple_of(i * 16, 16)
                xv = xb[pl.ds(o, 16)]
                yv = yb[pl.ds(o, 16)]
                zv = zb[pl.ds(o, 16)]
                valid = ((xv >= X_MIN) & (xv < X_MAX) &
                         (yv >= Y_MIN) & (yv < Y_MAX) &
                         (zv >= Z_MIN) & (zv < Z_MAX))
                xi = jnp.clip(((xv - X_MIN) / VX).astype(jnp.int32), 0, XS - 1)
                yi = jnp.clip(((yv - Y_MIN) / VY).astype(jnp.int32), 0, YS - 1)
                vox = jnp.where(valid, yi * XS + xi, NCELL + lane)
                pidx = ci * _ACH + i * 16 + lane
                plsc.store_scatter(win, [vox], pidx)
                return _

            lax.fori_loop(0, _ACH // 16, step, None)
        woff = pl.multiple_of(row * NWIN, 8)
        pltpu.sync_copy(win, shared_win.at[pl.ds(woff, NWIN)])

    @pl.when((s == 0) | (s == 8))
    def _():
        pl.run_scoped(phase_a,
                      pltpu.VMEM((NWIN,), jnp.int32),
                      pltpu.VMEM((_ACH,), jnp.float32),
                      pltpu.VMEM((_ACH,), jnp.float32),
                      pltpu.VMEM((_ACH,), jnp.float32))

    plsc.subcore_barrier()

    sl = s % 8
    cells0 = sl * (NCELL // 8)

    def phase_b(fbuf, wsl, obuf):
        pltpu.sync_copy(pts_ref.at[pl.ds(pl.multiple_of(pbase, 8), 5 * NPTS)],
                        fbuf)
        pltpu.sync_copy(
            shared_win.at[pl.ds(pl.multiple_of(row * NWIN + cells0, 8),
                                NCELL // 8)], wsl)
        for k in range(NCELL // 8 // _BCH):

            def step(i, _):
                o = pl.multiple_of(k * _BCH + i * 16, 16)
                w = wsl[pl.ds(o, 16)]
                valid = w >= 0
                wsafe = jnp.where(valid, w, lane)
                oo = pl.multiple_of(i * 16, 16)
                for f in range(5):
                    v = plsc.load_gather(fbuf, [wsafe + f * NPTS])
                    obuf[pl.ds(oo + f * _BCH, 16)] = jnp.where(valid, v, 0.0)
                return _

            lax.fori_loop(0, _BCH // 16, step, None)
            for f in range(5):
                dst = pl.multiple_of((b * 5 + f) * NCELL + cells0 + k * _BCH, 8)
                pltpu.sync_copy(
                    obuf.at[pl.ds(pl.multiple_of(f * _BCH, 8), _BCH)],
                    out_ref.at[pl.ds(dst, _BCH)])

    pl.run_scoped(phase_b,
                  pltpu.VMEM((5 * NPTS,), jnp.float32),
                  pltpu.VMEM((NCELL // 8,), jnp.int32),
                  pltpu.VMEM((5 * _BCH,), jnp.float32))


def _voxelize(radar):
    pts_t = radar.transpose(0, 2, 1).reshape(NB * 5 * NPTS)
    neg1 = jnp.full((NWIN,), -1, jnp.int32)
    mesh = plsc.VectorSubcoreMesh(core_axis_name="c", subcore_axis_name="s")
    grid = pl.kernel(
        _vox_body,
        mesh=mesh,
        out_type=jax.ShapeDtypeStruct((NB * 5 * NCELL,), jnp.float32),
        scratch_types=[pltpu.VMEM_SHARED((2 * NWIN,), jnp.int32)],
        compiler_params=pltpu.CompilerParams(needs_layout_passes=False),
    )(pts_t, neg1)
    return grid.reshape(NB, 5, YS, XS)


def _shift_w(x, dx):
    c, r = x.shape[0], x.shape[1]
    zcol = jnp.zeros((c, r, 1), jnp.float32)
    if dx == 0:
        return jnp.concatenate([zcol, x[:, :, :XS - 1]], axis=2)
    if dx == 1:
        return x
    return jnp.concatenate([x[:, :, 1:], zcol], axis=2)


_CK = 32


def _conv_body(x_ref, stats_ref, w_ref, b_ref, g_ref, be_ref,
               yc_ref, ostats_ref, xp_scr, stage_scr, sem, *,
               cin, cout, norm_in, rows):
    bi = pl.program_id(0)
    t = pl.program_id(1)

    @pl.when(t == 0)
    def _prologue():
        if norm_in:
            s1 = stats_ref[0, 0][:, None, None]
            s2 = stats_ref[0, 1][:, None, None]
            m = s1 / NCELL
            v = s2 / NCELL - m * m
            a = g_ref[...][:, :, None] * jax.lax.rsqrt(v + BN_EPS)
            c = be_ref[...][:, :, None] - m * a

        def _copy(i, buf):
            return pltpu.make_async_copy(
                x_ref.at[bi, :, pl.ds(i * _CK, _CK), :],
                stage_scr.at[buf], sem.at[buf])

        nck = YS // _CK
        _copy(0, 0).start()
        _copy(1, 1).start()
        for i in range(nck):
            _copy(i, i % 2).wait()
            xc = stage_scr[i % 2]
            if norm_in:
                xc = jnp.maximum(a * xc.astype(jnp.float32) + c, 0.0)
            xp_scr[:, 8 + i * _CK:8 + (i + 1) * _CK, :] = xc.astype(jnp.bfloat16)
            if i + 2 < nck:
                _copy(i + 2, i % 2).start()
        xp_scr[:, 0:8, :] = jnp.zeros((cin, 8, XS), jnp.bfloat16)
        xp_scr[:, YS + 8:YS + 16, :] = jnp.zeros((cin, 8, XS), jnp.bfloat16)

    xt = xp_scr[:, pl.ds(t * rows, rows + 16), :]
    xcat = jnp.concatenate(
        [xt[:, 7 + dy:7 + dy + rows, :] for dy in range(3)], axis=0)
    xs = xcat.reshape(3 * cin, rows * XS)
    acc = jnp.zeros((cout, rows, XS), jnp.float32)
    for dx in range(3):
        p = jax.lax.dot_general(
            w_ref[dx], xs, (((1,), (0,)), ((), ())),
            preferred_element_type=jnp.float32).reshape(cout, rows, XS)
        acc = acc + _shift_w(p, dx)
    acc = acc.reshape(cout, rows * XS) + b_ref[...]
    yc_ref[...] = acc.reshape(1, cout, rows, XS).astype(jnp.bfloat16)

    @pl.when(t == 0)
    def _init_stats():
        ostats_ref[...] = jnp.zeros((1, 2, cout), jnp.float32)

    ones = jnp.ones((rows * XS, 8), jnp.float32)
    ssum = jax.lax.dot_general(acc, ones, (((1,), (0,)), ((), ())),
                               preferred_element_type=jnp.float32)[:, 0]
    psq = jax.lax.dot_general(acc, acc, (((1,), (1,)), ((), ())),
                              preferred_element_type=jnp.float32)
    eye = (jax.lax.broadcasted_iota(jnp.int32, (cout, cout), 0) ==
           jax.lax.broadcasted_iota(jnp.int32, (cout, cout), 1))
    ssq = jnp.sum(jnp.where(eye, psq, 0.0), axis=1)
    ostats_ref[0, 0] = ostats_ref[0, 0] + ssum
    ostats_ref[0, 1] = ostats_ref[0, 1] + ssq


def _conv_layer(x, stats, w3, b, g, be, norm_in, rows=64):
    bsz, cin = x.shape[0], x.shape[1]
    cout = w3.shape[1]
    nt = YS // rows
    body = functools.partial(_conv_body, cin=cin, cout=cout,
                             norm_in=norm_in, rows=rows)
    return pl.pallas_call(
        body,
        grid=(bsz, nt),
        in_specs=[
            pl.BlockSpec(memory_space=pl.ANY),
            pl.BlockSpec((1, 2, cin), lambda i, t: (i, 0, 0)),
            pl.BlockSpec((3, cout, 3 * cin), lambda i, t: (0, 0, 0)),
            pl.BlockSpec((cout, 1), lambda i, t: (0, 0)),
            pl.BlockSpec((cin, 1), lambda i, t: (0, 0)),
            pl.BlockSpec((cin, 1), lambda i, t: (0, 0)),
        ],
        out_specs=[
            pl.BlockSpec((1, cout, rows, XS), lambda i, t: (i, 0, t, 0)),
            pl.BlockSpec((1, 2, cout), lambda i, t: (i, 0, 0)),
        ],
        out_shape=[
            jax.ShapeDtypeStruct((bsz, cout, YS, XS), jnp.bfloat16),
            jax.ShapeDtypeStruct((bsz, 2, cout), jnp.float32),
        ],
        scratch_shapes=[
            pltpu.VMEM((cin, YS + 16, XS), jnp.bfloat16),
            pltpu.VMEM((2, cin, _CK, XS), x.dtype),
            pltpu.SemaphoreType.DMA((2,)),
        ],
    )(x, stats, w3, b, g, be)


def _final_body(y_ref, stats_ref, g_ref, be_ref, o_ref):
    s1 = stats_ref[0, 0][:, None, None]
    s2 = stats_ref[0, 1][:, None, None]
    m = s1 / NCELL
    v = s2 / NCELL - m * m
    a = g_ref[...][:, :, None] * jax.lax.rsqrt(v + BN_EPS)
    c = be_ref[...][:, :, None] - m * a
    o_ref[0] = jnp.maximum(a * y_ref[0].astype(jnp.float32) + c, 0.0)


def _final_norm(y, stats, g, be, rows=64):
    bsz, cout = y.shape[0], y.shape[1]
    return pl.pallas_call(
        _final_body,
        grid=(bsz, YS // rows),
        in_specs=[
            pl.BlockSpec((1, cout, rows, XS), lambda i, t: (i, 0, t, 0)),
            pl.BlockSpec((1, 2, cout), lambda i, t: (i, 0, 0)),
            pl.BlockSpec((cout, 1), lambda i, t: (0, 0)),
            pl.BlockSpec((cout, 1), lambda i, t: (0, 0)),
        ],
        out_specs=pl.BlockSpec((1, cout, rows, XS), lambda i, t: (i, 0, t, 0)),
        out_shape=jax.ShapeDtypeStruct((bsz, cout, YS, XS), jnp.float32),
    )(y, stats, g, be)


def _w3(W):
    return W.transpose(3, 0, 2, 1).reshape(
        3, W.shape[0], 3 * W.shape[1]).astype(jnp.bfloat16)


def kernel(radar_points_list, W1, b1, g1, be1, W2, b2, g2, be2,
           W3, b3, g3, be3, W4, b4, g4, be4):
    grid = _voxelize(radar_points_list)
    bsz = grid.shape[0]
    dummy_stats = jnp.zeros((bsz, 2, 5), jnp.float32)
    dummy_gb = jnp.zeros((5, 1), jnp.float32)
    h, s = _conv_layer(grid, dummy_stats, _w3(W1), b1[:, None],
                       dummy_gb, dummy_gb, norm_in=False)
    for (W, b, g, be, gp, bep) in ((W2, b2, g2, be2, g1, be1),
                                   (W3, b3, g3, be3, g2, be2),
                                   (W4, b4, g4, be4, g3, be3)):
        h, s_next = _conv_layer(h, s, _w3(W), b[:, None],
                                gp[:, None], bep[:, None], norm_in=True)
        s = s_next
    return _final_norm(h, s, g4[:, None], be4[:, None])


# speedup vs baseline: 1.6045x; 1.4166x over previous
import functools

import jax
import jax.numpy as jnp
from jax import lax
from jax.experimental import pallas as pl
from jax.experimental.pallas import tpu as pltpu
from jax.experimental.pallas import tpu_sc as plsc

X_MIN, Y_MIN, Z_MIN = -51.2, -51.2, -5.0
X_MAX, Y_MAX, Z_MAX = 51.2, 51.2, 3.0
VX, VY = 0.4, 0.4
XS = int((X_MAX - X_MIN) / VX)
YS = int((Y_MAX - Y_MIN) / VY)
NCELL = YS * XS
BN_EPS = 1e-5


NB = 4
NPTS = 20000
NWIN = NCELL + 16
_ACH = 2000
_BCH = 1024


def _vox_body(pts_ref, neg1_ref, out_ref, shared_win):
    c = lax.axis_index("c")
    s = lax.axis_index("s")
    lane = lax.iota(jnp.int32, 16)
    row = s // 8
    b = c * 2 + row
    pbase = b * 5 * NPTS

    def phase_a(win, xb, yb, zb):
        pltpu.sync_copy(neg1_ref, win)
        for ci in range(NPTS // _ACH):
            for buf, f in ((xb, 0), (yb, 1), (zb, 2)):
                off = pl.multiple_of(pbase + f * NPTS + ci * _ACH, 8)
                pltpu.sync_copy(pts_ref.at[pl.ds(off, _ACH)], buf)

            def step(i, _):
                o = pl.multiple_of(i * 16, 16)
                xv = xb[pl.ds(o, 16)]
                yv = yb[pl.ds(o, 16)]
                zv = zb[pl.ds(o, 16)]
                valid = ((xv >= X_MIN) & (xv < X_MAX) &
                         (yv >= Y_MIN) & (yv < Y_MAX) &
                         (zv >= Z_MIN) & (zv < Z_MAX))
                xi = jnp.clip(((xv - X_MIN) / VX).astype(jnp.int32), 0, XS - 1)
                yi = jnp.clip(((yv - Y_MIN) / VY).astype(jnp.int32), 0, YS - 1)
                vox = jnp.where(valid, yi * XS + xi, NCELL + lane)
                pidx = ci * _ACH + i * 16 + lane
                plsc.store_scatter(win, [vox], pidx)
                return _

            lax.fori_loop(0, _ACH // 16, step, None)
        woff = pl.multiple_of(row * NWIN, 8)
        pltpu.sync_copy(win, shared_win.at[pl.ds(woff, NWIN)])

    @pl.when((s == 0) | (s == 8))
    def _():
        pl.run_scoped(phase_a,
                      pltpu.VMEM((NWIN,), jnp.int32),
                      pltpu.VMEM((_ACH,), jnp.float32),
                      pltpu.VMEM((_ACH,), jnp.float32),
                      pltpu.VMEM((_ACH,), jnp.float32))

    plsc.subcore_barrier()

    sl = s % 8
    cells0 = sl * (NCELL // 8)

    def phase_b(fbuf, wsl, obuf):
        pltpu.sync_copy(pts_ref.at[pl.ds(pl.multiple_of(pbase, 8), 5 * NPTS)],
                        fbuf)
        pltpu.sync_copy(
            shared_win.at[pl.ds(pl.multiple_of(row * NWIN + cells0, 8),
                                NCELL // 8)], wsl)
        for k in range(NCELL // 8 // _BCH):

            def step(i, _):
                o = pl.multiple_of(k * _BCH + i * 16, 16)
                w = wsl[pl.ds(o, 16)]
                valid = w >= 0
                wsafe = jnp.where(valid, w, lane)
                oo = pl.multiple_of(i * 16, 16)
                for f in range(5):
                    v = plsc.load_gather(fbuf, [wsafe + f * NPTS])
                    obuf[pl.ds(oo + f * _BCH, 16)] = jnp.where(valid, v, 0.0)
                return _

            lax.fori_loop(0, _BCH // 16, step, None)
            for f in range(5):
                dst = pl.multiple_of((b * 5 + f) * NCELL + cells0 + k * _BCH, 8)
                pltpu.sync_copy(
                    obuf.at[pl.ds(pl.multiple_of(f * _BCH, 8), _BCH)],
                    out_ref.at[pl.ds(dst, _BCH)])

    pl.run_scoped(phase_b,
                  pltpu.VMEM((5 * NPTS,), jnp.float32),
                  pltpu.VMEM((NCELL // 8,), jnp.int32),
                  pltpu.VMEM((5 * _BCH,), jnp.float32))


def _voxelize(radar):
    pts_t = radar.transpose(0, 2, 1).reshape(NB * 5 * NPTS)
    neg1 = jnp.full((NWIN,), -1, jnp.int32)
    mesh = plsc.VectorSubcoreMesh(core_axis_name="c", subcore_axis_name="s")
    grid = pl.kernel(
        _vox_body,
        mesh=mesh,
        out_type=jax.ShapeDtypeStruct((NB * 5 * NCELL,), jnp.float32),
        scratch_types=[pltpu.VMEM_SHARED((2 * NWIN,), jnp.int32)],
        compiler_params=pltpu.CompilerParams(needs_layout_passes=False),
    )(pts_t, neg1)
    return grid.reshape(NB, 5, NCELL)


def _shift_flat(p, dx):
    if dx == 1:
        return p
    n = p.shape[1]
    wpos = jax.lax.broadcasted_iota(jnp.int32, p.shape, 1) % XS
    zcol = jnp.zeros((p.shape[0], 1), p.dtype)
    if dx == 0:
        shifted = jnp.concatenate([zcol, p[:, :n - 1]], axis=1)
        return jnp.where(wpos == 0, 0.0, shifted)
    shifted = jnp.concatenate([p[:, 1:], zcol], axis=1)
    return jnp.where(wpos == XS - 1, 0.0, shifted)


_CK = 32


def _conv_body(x_ref, stats_ref, w_ref, b_ref, g_ref, be_ref,
               yc_ref, ostats_ref, xp_scr, stage_scr, sem, *,
               cin, cout, norm_in, rows):
    bi = pl.program_id(0)
    t = pl.program_id(1)

    @pl.when(t == 0)
    def _prologue():
        if norm_in:
            s1 = stats_ref[0, 0][:, None]
            s2 = stats_ref[0, 1][:, None]
            m = s1 / NCELL
            v = s2 / NCELL - m * m
            a = g_ref[...] * jax.lax.rsqrt(v + BN_EPS)
            c = be_ref[...] - m * a

        def _copy(i, buf):
            return pltpu.make_async_copy(
                x_ref.at[bi, :, pl.ds(i * _CK * XS, _CK * XS)],
                stage_scr.at[buf], sem.at[buf])

        nck = YS // _CK
        _copy(0, 0).start()
        _copy(1, 1).start()
        for i in range(nck):
            _copy(i, i % 2).wait()
            xc = stage_scr[i % 2]
            if norm_in:
                xc = jnp.maximum(a * xc.astype(jnp.float32) + c, 0.0)
            xp_scr[:, (8 + i * _CK) * XS:(8 + (i + 1) * _CK) * XS] = (
                xc.astype(jnp.bfloat16))
            if i + 2 < nck:
                _copy(i + 2, i % 2).start()
        xp_scr[:, 0:8 * XS] = jnp.zeros((cin, 8 * XS), jnp.bfloat16)
        xp_scr[:, (YS + 8) * XS:(YS + 16) * XS] = (
            jnp.zeros((cin, 8 * XS), jnp.bfloat16))

    xt = xp_scr[:, pl.ds(pl.multiple_of(t * rows * XS, 256),
                         (rows + 16) * XS)]
    xcat = jnp.concatenate(
        [xt[:, (7 + dy) * XS:(7 + dy + rows) * XS] for dy in range(3)],
        axis=0)
    acc = jnp.zeros((cout, rows * XS), jnp.float32)
    for dx in range(3):
        p = jax.lax.dot_general(
            w_ref[dx], xcat, (((1,), (0,)), ((), ())),
            preferred_element_type=jnp.float32)
        acc = acc + _shift_flat(p, dx)
    acc = acc + b_ref[...]
    yc_ref[...] = acc[None].astype(jnp.bfloat16)

    @pl.when(t == 0)
    def _init_stats():
        ostats_ref[...] = jnp.zeros((1, 2, cout), jnp.float32)

    ones = jnp.ones((rows * XS, 8), jnp.float32)
    ssum = jax.lax.dot_general(acc, ones, (((1,), (0,)), ((), ())),
                               preferred_element_type=jnp.float32)[:, 0]
    psq = jax.lax.dot_general(acc, acc, (((1,), (1,)), ((), ())),
                              preferred_element_type=jnp.float32)
    eye = (jax.lax.broadcasted_iota(jnp.int32, (cout, cout), 0) ==
           jax.lax.broadcasted_iota(jnp.int32, (cout, cout), 1))
    ssq = jnp.sum(jnp.where(eye, psq, 0.0), axis=1)
    ostats_ref[0, 0] = ostats_ref[0, 0] + ssum
    ostats_ref[0, 1] = ostats_ref[0, 1] + ssq


def _conv_layer(x, stats, w3, b, g, be, norm_in, rows=64):
    bsz, cin = x.shape[0], x.shape[1]
    cout = w3.shape[1]
    nt = YS // rows
    body = functools.partial(_conv_body, cin=cin, cout=cout,
                             norm_in=norm_in, rows=rows)
    return pl.pallas_call(
        body,
        grid=(bsz, nt),
        in_specs=[
            pl.BlockSpec(memory_space=pl.ANY),
            pl.BlockSpec((1, 2, cin), lambda i, t: (i, 0, 0)),
            pl.BlockSpec((3, cout, 3 * cin), lambda i, t: (0, 0, 0)),
            pl.BlockSpec((cout, 1), lambda i, t: (0, 0)),
            pl.BlockSpec((cin, 1), lambda i, t: (0, 0)),
            pl.BlockSpec((cin, 1), lambda i, t: (0, 0)),
        ],
        out_specs=[
            pl.BlockSpec((1, cout, rows * XS), lambda i, t: (i, 0, t)),
            pl.BlockSpec((1, 2, cout), lambda i, t: (i, 0, 0)),
        ],
        out_shape=[
            jax.ShapeDtypeStruct((bsz, cout, NCELL), jnp.bfloat16),
            jax.ShapeDtypeStruct((bsz, 2, cout), jnp.float32),
        ],
        scratch_shapes=[
            pltpu.VMEM((cin, (YS + 16) * XS), jnp.bfloat16),
            pltpu.VMEM((2, cin, _CK * XS), x.dtype),
            pltpu.SemaphoreType.DMA((2,)),
        ],
    )(x, stats, w3, b, g, be)


def _final_body(y_ref, stats_ref, g_ref, be_ref, o_ref):
    s1 = stats_ref[0, 0][:, None]
    s2 = stats_ref[0, 1][:, None]
    m = s1 / NCELL
    v = s2 / NCELL - m * m
    a = g_ref[...] * jax.lax.rsqrt(v + BN_EPS)
    c = be_ref[...] - m * a
    o_ref[0] = jnp.maximum(a * y_ref[0].astype(jnp.float32) + c, 0.0)


def _final_norm(y, stats, g, be, rows=64):
    bsz, cout = y.shape[0], y.shape[1]
    return pl.pallas_call(
        _final_body,
        grid=(bsz, YS // rows),
        in_specs=[
            pl.BlockSpec((1, cout, rows * XS), lambda i, t: (i, 0, t)),
            pl.BlockSpec((1, 2, cout), lambda i, t: (i, 0, 0)),
            pl.BlockSpec((cout, 1), lambda i, t: (0, 0)),
            pl.BlockSpec((cout, 1), lambda i, t: (0, 0)),
        ],
        out_specs=pl.BlockSpec((1, cout, rows * XS), lambda i, t: (i, 0, t)),
        out_shape=jax.ShapeDtypeStruct((bsz, cout, NCELL), jnp.float32),
    )(y, stats, g, be)


def _w3(W):
    return W.transpose(3, 0, 2, 1).reshape(
        3, W.shape[0], 3 * W.shape[1]).astype(jnp.bfloat16)


def kernel(radar_points_list, W1, b1, g1, be1, W2, b2, g2, be2,
           W3, b3, g3, be3, W4, b4, g4, be4):
    grid = _voxelize(radar_points_list)
    bsz = grid.shape[0]
    dummy_stats = jnp.zeros((bsz, 2, 5), jnp.float32)
    dummy_gb = jnp.zeros((5, 1), jnp.float32)
    h, s = _conv_layer(grid, dummy_stats, _w3(W1), b1[:, None],
                       dummy_gb, dummy_gb, norm_in=False)
    for (W, b, g, be, gp, bep) in ((W2, b2, g2, be2, g1, be1),
                                   (W3, b3, g3, be3, g2, be2),
                                   (W4, b4, g4, be4, g3, be3)):
        h, s_next = _conv_layer(h, s, _w3(W), b[:, None],
                                gp[:, None], bep[:, None], norm_in=True)
        s = s_next
    out = _final_norm(h, s, g4[:, None], be4[:, None])
    return out.reshape(out.shape[0], out.shape[1], YS, XS)


# speedup vs baseline: 1.6790x; 1.0464x over previous
import functools

import jax
import jax.numpy as jnp
from jax import lax
from jax.experimental import pallas as pl
from jax.experimental.pallas import tpu as pltpu
from jax.experimental.pallas import tpu_sc as plsc

X_MIN, Y_MIN, Z_MIN = -51.2, -51.2, -5.0
X_MAX, Y_MAX, Z_MAX = 51.2, 51.2, 3.0
VX, VY = 0.4, 0.4
XS = int((X_MAX - X_MIN) / VX)
YS = int((Y_MAX - Y_MIN) / VY)
NCELL = YS * XS
BN_EPS = 1e-5


NB = 4
NPTS = 20000
NWIN = NCELL + 16
_ACH = 2000
_BCH = 1024


def _vox_body(pts_ref, neg1_ref, out_ref, shared_win):
    c = lax.axis_index("c")
    s = lax.axis_index("s")
    lane = lax.iota(jnp.int32, 16)
    row = s // 8
    b = c * 2 + row
    pbase = b * 5 * NPTS

    def phase_a(win, xb, yb, zb):
        pltpu.sync_copy(neg1_ref, win)
        for ci in range(NPTS // _ACH):
            for buf, f in ((xb, 0), (yb, 1), (zb, 2)):
                off = pl.multiple_of(pbase + f * NPTS + ci * _ACH, 8)
                pltpu.sync_copy(pts_ref.at[pl.ds(off, _ACH)], buf)

            def step(i, _):
                o = pl.multiple_of(i * 16, 16)
                xv = xb[pl.ds(o, 16)]
                yv = yb[pl.ds(o, 16)]
                zv = zb[pl.ds(o, 16)]
                valid = ((xv >= X_MIN) & (xv < X_MAX) &
                         (yv >= Y_MIN) & (yv < Y_MAX) &
                         (zv >= Z_MIN) & (zv < Z_MAX))
                xi = jnp.clip(((xv - X_MIN) / VX).astype(jnp.int32), 0, XS - 1)
                yi = jnp.clip(((yv - Y_MIN) / VY).astype(jnp.int32), 0, YS - 1)
                vox = jnp.where(valid, yi * XS + xi, NCELL + lane)
                pidx = ci * _ACH + i * 16 + lane
                plsc.store_scatter(win, [vox], pidx)
                return _

            lax.fori_loop(0, _ACH // 16, step, None)
        woff = pl.multiple_of(row * NWIN, 8)
        pltpu.sync_copy(win, shared_win.at[pl.ds(woff, NWIN)])

    @pl.when((s == 0) | (s == 8))
    def _():
        pl.run_scoped(phase_a,
                      pltpu.VMEM((NWIN,), jnp.int32),
                      pltpu.VMEM((_ACH,), jnp.float32),
                      pltpu.VMEM((_ACH,), jnp.float32),
                      pltpu.VMEM((_ACH,), jnp.float32))

    plsc.subcore_barrier()

    sl = s % 8
    cells0 = sl * (NCELL // 8)

    def phase_b(fbuf, wsl, obuf):
        pltpu.sync_copy(pts_ref.at[pl.ds(pl.multiple_of(pbase, 8), 5 * NPTS)],
                        fbuf)
        pltpu.sync_copy(
            shared_win.at[pl.ds(pl.multiple_of(row * NWIN + cells0, 8),
                                NCELL // 8)], wsl)
        for k in range(NCELL // 8 // _BCH):

            def step(i, _):
                o = pl.multiple_of(k * _BCH + i * 16, 16)
                w = wsl[pl.ds(o, 16)]
                valid = w >= 0
                wsafe = jnp.where(valid, w, lane)
                oo = pl.multiple_of(i * 16, 16)
                for f in range(5):
                    v = plsc.load_gather(fbuf, [wsafe + f * NPTS])
                    obuf[pl.ds(oo + f * _BCH, 16)] = jnp.where(valid, v, 0.0)
                return _

            lax.fori_loop(0, _BCH // 16, step, None)
            for f in range(5):
                dst = pl.multiple_of((b * 5 + f) * NCELL + cells0 + k * _BCH, 8)
                pltpu.sync_copy(
                    obuf.at[pl.ds(pl.multiple_of(f * _BCH, 8), _BCH)],
                    out_ref.at[pl.ds(dst, _BCH)])

    pl.run_scoped(phase_b,
                  pltpu.VMEM((5 * NPTS,), jnp.float32),
                  pltpu.VMEM((NCELL // 8,), jnp.int32),
                  pltpu.VMEM((5 * _BCH,), jnp.float32))


def _voxelize(radar):
    pts_t = radar.transpose(0, 2, 1).reshape(NB * 5 * NPTS)
    neg1 = jnp.full((NWIN,), -1, jnp.int32)
    mesh = plsc.VectorSubcoreMesh(core_axis_name="c", subcore_axis_name="s")
    grid = pl.kernel(
        _vox_body,
        mesh=mesh,
        out_type=jax.ShapeDtypeStruct((NB * 5 * NCELL,), jnp.float32),
        scratch_types=[pltpu.VMEM_SHARED((2 * NWIN,), jnp.int32)],
        compiler_params=pltpu.CompilerParams(needs_layout_passes=False),
    )(pts_t, neg1)
    return grid.reshape(NB, 5, NCELL)


def _shift_flat(p, dx):
    if dx == 1:
        return p
    n = p.shape[1]
    wpos = jax.lax.broadcasted_iota(jnp.int32, p.shape, 1) % XS
    zcol = jnp.zeros((p.shape[0], 1), p.dtype)
    if dx == 0:
        shifted = jnp.concatenate([zcol, p[:, :n - 1]], axis=1)
        return jnp.where(wpos == 0, 0.0, shifted)
    shifted = jnp.concatenate([p[:, 1:], zcol], axis=1)
    return jnp.where(wpos == XS - 1, 0.0, shifted)


_CK = 32


def _conv_body(x_ref, stats_ref, w_ref, b_ref, g_ref, be_ref,
               yc_ref, ostats_ref, xp_scr, stage_scr, sem, *,
               cin, cout, norm_in, rows):
    bi = pl.program_id(0)
    t = pl.program_id(1)

    @pl.when(t == 0)
    def _prologue():
        if norm_in:
            s1 = stats_ref[0, 0][:, None]
            s2 = stats_ref[0, 1][:, None]
            m = s1 / NCELL
            v = s2 / NCELL - m * m
            a = g_ref[...] * jax.lax.rsqrt(v + BN_EPS)
            c = be_ref[...] - m * a

        def _copy(i, buf):
            return pltpu.make_async_copy(
                x_ref.at[bi, :, pl.ds(i * _CK * XS, _CK * XS)],
                stage_scr.at[buf], sem.at[buf])

        nck = YS // _CK
        _copy(0, 0).start()
        _copy(1, 1).start()
        for i in range(nck):
            _copy(i, i % 2).wait()
            xc = stage_scr[i % 2]
            if norm_in:
                xc = jnp.maximum(a * xc.astype(jnp.float32) + c, 0.0)
            xp_scr[:, (8 + i * _CK) * XS:(8 + (i + 1) * _CK) * XS] = (
                xc.astype(jnp.bfloat16))
            if i + 2 < nck:
                _copy(i + 2, i % 2).start()
        xp_scr[:, 0:8 * XS] = jnp.zeros((cin, 8 * XS), jnp.bfloat16)
        xp_scr[:, (YS + 8) * XS:(YS + 16) * XS] = (
            jnp.zeros((cin, 8 * XS), jnp.bfloat16))

    xt = xp_scr[:, pl.ds(pl.multiple_of(t * rows * XS, 256),
                         (rows + 16) * XS)]
    xcat = jnp.concatenate(
        [xt[:, (7 + dy) * XS:(7 + dy + rows) * XS] for dy in range(3)],
        axis=0)
    acc = jnp.zeros((cout, rows * XS), jnp.float32)
    for dx in range(3):
        p = jax.lax.dot_general(
            w_ref[dx], xcat, (((1,), (0,)), ((), ())),
            preferred_element_type=jnp.float32)
        acc = acc + _shift_flat(p, dx)
    acc = acc + b_ref[...]
    yc_ref[...] = acc[None].astype(jnp.bfloat16)

    @pl.when(t == 0)
    def _init_stats():
        ostats_ref[...] = jnp.zeros((1, 2, cout), jnp.float32)

    ones = jnp.ones((rows * XS, 8), jnp.float32)
    both = jnp.concatenate([acc, acc * acc], axis=0)
    r = jax.lax.dot_general(both, ones, (((1,), (0,)), ((), ())),
                            preferred_element_type=jnp.float32)[:, 0]
    ostats_ref[0, 0] = ostats_ref[0, 0] + r[:cout]
    ostats_ref[0, 1] = ostats_ref[0, 1] + r[cout:]


def _conv_layer(x, stats, w3, b, g, be, norm_in, rows=64):
    bsz, cin = x.shape[0], x.shape[1]
    cout = w3.shape[1]
    nt = YS // rows
    body = functools.partial(_conv_body, cin=cin, cout=cout,
                             norm_in=norm_in, rows=rows)
    return pl.pallas_call(
        body,
        grid=(bsz, nt),
        in_specs=[
            pl.BlockSpec(memory_space=pl.ANY),
            pl.BlockSpec((1, 2, cin), lambda i, t: (i, 0, 0)),
            pl.BlockSpec((3, cout, 3 * cin), lambda i, t: (0, 0, 0)),
            pl.BlockSpec((cout, 1), lambda i, t: (0, 0)),
            pl.BlockSpec((cin, 1), lambda i, t: (0, 0)),
            pl.BlockSpec((cin, 1), lambda i, t: (0, 0)),
        ],
        out_specs=[
            pl.BlockSpec((1, cout, rows * XS), lambda i, t: (i, 0, t)),
            pl.BlockSpec((1, 2, cout), lambda i, t: (i, 0, 0)),
        ],
        out_shape=[
            jax.ShapeDtypeStruct((bsz, cout, NCELL), jnp.bfloat16),
            jax.ShapeDtypeStruct((bsz, 2, cout), jnp.float32),
        ],
        scratch_shapes=[
            pltpu.VMEM((cin, (YS + 16) * XS), jnp.bfloat16),
            pltpu.VMEM((2, cin, _CK * XS), x.dtype),
            pltpu.SemaphoreType.DMA((2,)),
        ],
    )(x, stats, w3, b, g, be)


def _final_body(y_ref, stats_ref, g_ref, be_ref, o_ref):
    s1 = stats_ref[0, 0][:, None]
    s2 = stats_ref[0, 1][:, None]
    m = s1 / NCELL
    v = s2 / NCELL - m * m
    a = g_ref[...] * jax.lax.rsqrt(v + BN_EPS)
    c = be_ref[...] - m * a
    o_ref[0] = jnp.maximum(a * y_ref[0].astype(jnp.float32) + c, 0.0)


def _final_norm(y, stats, g, be, rows=64):
    bsz, cout = y.shape[0], y.shape[1]
    return pl.pallas_call(
        _final_body,
        grid=(bsz, YS // rows),
        in_specs=[
            pl.BlockSpec((1, cout, rows * XS), lambda i, t: (i, 0, t)),
            pl.BlockSpec((1, 2, cout), lambda i, t: (i, 0, 0)),
            pl.BlockSpec((cout, 1), lambda i, t: (0, 0)),
            pl.BlockSpec((cout, 1), lambda i, t: (0, 0)),
        ],
        out_specs=pl.BlockSpec((1, cout, rows * XS), lambda i, t: (i, 0, t)),
        out_shape=jax.ShapeDtypeStruct((bsz, cout, NCELL), jnp.float32),
    )(y, stats, g, be)


def _w3(W):
    return W.transpose(3, 0, 2, 1).reshape(
        3, W.shape[0], 3 * W.shape[1]).astype(jnp.bfloat16)


def kernel(radar_points_list, W1, b1, g1, be1, W2, b2, g2, be2,
           W3, b3, g3, be3, W4, b4, g4, be4):
    grid = _voxelize(radar_points_list)
    bsz = grid.shape[0]
    dummy_stats = jnp.zeros((bsz, 2, 5), jnp.float32)
    dummy_gb = jnp.zeros((5, 1), jnp.float32)
    h, s = _conv_layer(grid, dummy_stats, _w3(W1), b1[:, None],
                       dummy_gb, dummy_gb, norm_in=False)
    for (W, b, g, be, gp, bep) in ((W2, b2, g2, be2, g1, be1),
                                   (W3, b3, g3, be3, g2, be2),
                                   (W4, b4, g4, be4, g3, be3)):
        h, s_next = _conv_layer(h, s, _w3(W), b[:, None],
                                gp[:, None], bep[:, None], norm_in=True)
        s = s_next
    out = _final_norm(h, s, g4[:, None], be4[:, None])
    return out.reshape(out.shape[0], out.shape[1], YS, XS)


# speedup vs baseline: 1.7147x; 1.0212x over previous
import functools

import jax
import jax.numpy as jnp
from jax import lax
from jax.experimental import pallas as pl
from jax.experimental.pallas import tpu as pltpu
from jax.experimental.pallas import tpu_sc as plsc

X_MIN, Y_MIN, Z_MIN = -51.2, -51.2, -5.0
X_MAX, Y_MAX, Z_MAX = 51.2, 51.2, 3.0
VX, VY = 0.4, 0.4
XS = int((X_MAX - X_MIN) / VX)
YS = int((Y_MAX - Y_MIN) / VY)
NCELL = YS * XS
BN_EPS = 1e-5


NB = 4
NPTS = 20000
NWIN = NCELL + 16
_ACH = 2000
_BCH = 1024


def _vox_body(pts_ref, neg1_ref, out_ref, shared_win):
    c = lax.axis_index("c")
    s = lax.axis_index("s")
    lane = lax.iota(jnp.int32, 16)
    row = s // 8
    b = c * 2 + row
    pbase = b * 5 * NPTS

    def phase_a(win, xb, yb, zb):
        pltpu.sync_copy(neg1_ref, win)
        for ci in range(NPTS // _ACH):
            for buf, f in ((xb, 0), (yb, 1), (zb, 2)):
                off = pl.multiple_of(pbase + f * NPTS + ci * _ACH, 8)
                pltpu.sync_copy(pts_ref.at[pl.ds(off, _ACH)], buf)

            def step(i, _):
                o = pl.multiple_of(i * 16, 16)
                xv = xb[pl.ds(o, 16)]
                yv = yb[pl.ds(o, 16)]
                zv = zb[pl.ds(o, 16)]
                valid = ((xv >= X_MIN) & (xv < X_MAX) &
                         (yv >= Y_MIN) & (yv < Y_MAX) &
                         (zv >= Z_MIN) & (zv < Z_MAX))
                xi = jnp.clip(((xv - X_MIN) / VX).astype(jnp.int32), 0, XS - 1)
                yi = jnp.clip(((yv - Y_MIN) / VY).astype(jnp.int32), 0, YS - 1)
                vox = jnp.where(valid, yi * XS + xi, NCELL + lane)
                pidx = ci * _ACH + i * 16 + lane
                plsc.store_scatter(win, [vox], pidx)
                return _

            lax.fori_loop(0, _ACH // 16, step, None)
        woff = pl.multiple_of(row * NWIN, 8)
        pltpu.sync_copy(win, shared_win.at[pl.ds(woff, NWIN)])

    @pl.when((s == 0) | (s == 8))
    def _():
        pl.run_scoped(phase_a,
                      pltpu.VMEM((NWIN,), jnp.int32),
                      pltpu.VMEM((_ACH,), jnp.float32),
                      pltpu.VMEM((_ACH,), jnp.float32),
                      pltpu.VMEM((_ACH,), jnp.float32))

    plsc.subcore_barrier()

    sl = s % 8
    cells0 = sl * (NCELL // 8)

    def phase_b(fbuf, wsl, obuf):
        pltpu.sync_copy(pts_ref.at[pl.ds(pl.multiple_of(pbase, 8), 5 * NPTS)],
                        fbuf)
        pltpu.sync_copy(
            shared_win.at[pl.ds(pl.multiple_of(row * NWIN + cells0, 8),
                                NCELL // 8)], wsl)
        for k in range(NCELL // 8 // _BCH):

            def step(i, _):
                o = pl.multiple_of(k * _BCH + i * 16, 16)
                w = wsl[pl.ds(o, 16)]
                valid = w >= 0
                wsafe = jnp.where(valid, w, lane)
                oo = pl.multiple_of(i * 16, 16)
                for f in range(5):
                    v = plsc.load_gather(fbuf, [wsafe + f * NPTS])
                    obuf[pl.ds(oo + f * _BCH, 16)] = jnp.where(valid, v, 0.0)
                return _

            lax.fori_loop(0, _BCH // 16, step, None)
            for f in range(5):
                dst = pl.multiple_of((b * 5 + f) * NCELL + cells0 + k * _BCH, 8)
                pltpu.sync_copy(
                    obuf.at[pl.ds(pl.multiple_of(f * _BCH, 8), _BCH)],
                    out_ref.at[pl.ds(dst, _BCH)])

    pl.run_scoped(phase_b,
                  pltpu.VMEM((5 * NPTS,), jnp.float32),
                  pltpu.VMEM((NCELL // 8,), jnp.int32),
                  pltpu.VMEM((5 * _BCH,), jnp.float32))


def _voxelize(radar):
    pts_t = radar.transpose(0, 2, 1).reshape(NB * 5 * NPTS)
    neg1 = jnp.full((NWIN,), -1, jnp.int32)
    mesh = plsc.VectorSubcoreMesh(core_axis_name="c", subcore_axis_name="s")
    grid = pl.kernel(
        _vox_body,
        mesh=mesh,
        out_type=jax.ShapeDtypeStruct((NB * 5 * NCELL,), jnp.float32),
        scratch_types=[pltpu.VMEM_SHARED((2 * NWIN,), jnp.int32)],
        compiler_params=pltpu.CompilerParams(needs_layout_passes=False),
    )(pts_t, neg1)
    return grid.reshape(NB, 5, NCELL)


def _shift_flat(p, dx):
    if dx == 1:
        return p
    n = p.shape[1]
    wpos = jax.lax.broadcasted_iota(jnp.int32, p.shape, 1) % XS
    zcol = jnp.zeros((p.shape[0], 1), p.dtype)
    if dx == 0:
        shifted = jnp.concatenate([zcol, p[:, :n - 1]], axis=1)
        return jnp.where(wpos == 0, 0.0, shifted)
    shifted = jnp.concatenate([p[:, 1:], zcol], axis=1)
    return jnp.where(wpos == XS - 1, 0.0, shifted)


_CK = 32


def _conv_body(x_ref, stats_ref, w_ref, b_ref, g_ref, be_ref,
               yc_ref, ostats_ref, xp_scr, stage_scr, sem, *,
               cin, cout, norm_in, rows):
    bi = pl.program_id(0)
    t = pl.program_id(1)

    @pl.when(t == 0)
    def _prologue():
        if norm_in:
            s1 = stats_ref[0, 0][:, None]
            s2 = stats_ref[0, 1][:, None]
            m = s1 / NCELL
            v = s2 / NCELL - m * m
            a = g_ref[...] * jax.lax.rsqrt(v + BN_EPS)
            c = be_ref[...] - m * a

        def _copy(i, buf):
            return pltpu.make_async_copy(
                x_ref.at[bi, :, pl.ds(i * _CK * XS, _CK * XS)],
                stage_scr.at[buf], sem.at[buf])

        nck = YS // _CK
        _copy(0, 0).start()
        _copy(1, 1).start()
        for i in range(nck):
            _copy(i, i % 2).wait()
            xc = stage_scr[i % 2]
            if norm_in:
                xc = jnp.maximum(a * xc.astype(jnp.float32) + c, 0.0)
            xp_scr[:, (8 + i * _CK) * XS:(8 + (i + 1) * _CK) * XS] = (
                xc.astype(jnp.bfloat16))
            if i + 2 < nck:
                _copy(i + 2, i % 2).start()
        xp_scr[:, 0:8 * XS] = jnp.zeros((cin, 8 * XS), jnp.bfloat16)
        xp_scr[:, (YS + 8) * XS:(YS + 16) * XS] = (
            jnp.zeros((cin, 8 * XS), jnp.bfloat16))

    xt = xp_scr[:, pl.ds(pl.multiple_of(t * rows * XS, 256),
                         (rows + 16) * XS)]
    xcat = jnp.concatenate(
        [xt[:, (7 + dy) * XS:(7 + dy + rows) * XS] for dy in range(3)],
        axis=0)
    acc = jnp.zeros((cout, rows * XS), jnp.float32)
    for dx in range(3):
        p = jax.lax.dot_general(
            w_ref[dx], xcat, (((1,), (0,)), ((), ())),
            preferred_element_type=jnp.float32)
        acc = acc + _shift_flat(p, dx)
    acc = acc + b_ref[...]
    yc_ref[...] = acc[None].astype(jnp.bfloat16)

    @pl.when(t == 0)
    def _init_stats():
        ostats_ref[...] = jnp.zeros((1, 2, cout), jnp.float32)

    ones = jnp.ones((rows * XS, 8), jnp.float32)
    both = jnp.concatenate([acc, acc * acc], axis=0)
    r = jax.lax.dot_general(both, ones, (((1,), (0,)), ((), ())),
                            preferred_element_type=jnp.float32)[:, 0]
    ostats_ref[0, 0] = ostats_ref[0, 0] + r[:cout]
    ostats_ref[0, 1] = ostats_ref[0, 1] + r[cout:]


def _conv_layer(x, stats, w3, b, g, be, norm_in, rows=128):
    bsz, cin = x.shape[0], x.shape[1]
    cout = w3.shape[1]
    nt = YS // rows
    body = functools.partial(_conv_body, cin=cin, cout=cout,
                             norm_in=norm_in, rows=rows)
    return pl.pallas_call(
        body,
        grid=(bsz, nt),
        in_specs=[
            pl.BlockSpec(memory_space=pl.ANY),
            pl.BlockSpec((1, 2, cin), lambda i, t: (i, 0, 0)),
            pl.BlockSpec((3, cout, 3 * cin), lambda i, t: (0, 0, 0)),
            pl.BlockSpec((cout, 1), lambda i, t: (0, 0)),
            pl.BlockSpec((cin, 1), lambda i, t: (0, 0)),
            pl.BlockSpec((cin, 1), lambda i, t: (0, 0)),
        ],
        out_specs=[
            pl.BlockSpec((1, cout, rows * XS), lambda i, t: (i, 0, t)),
            pl.BlockSpec((1, 2, cout), lambda i, t: (i, 0, 0)),
        ],
        out_shape=[
            jax.ShapeDtypeStruct((bsz, cout, NCELL), jnp.bfloat16),
            jax.ShapeDtypeStruct((bsz, 2, cout), jnp.float32),
        ],
        scratch_shapes=[
            pltpu.VMEM((cin, (YS + 16) * XS), jnp.bfloat16),
            pltpu.VMEM((2, cin, _CK * XS), x.dtype),
            pltpu.SemaphoreType.DMA((2,)),
        ],
    )(x, stats, w3, b, g, be)


def _final_body(y_ref, stats_ref, g_ref, be_ref, o_ref):
    s1 = stats_ref[0, 0][:, None]
    s2 = stats_ref[0, 1][:, None]
    m = s1 / NCELL
    v = s2 / NCELL - m * m
    a = g_ref[...] * jax.lax.rsqrt(v + BN_EPS)
    c = be_ref[...] - m * a
    o_ref[0] = jnp.maximum(a * y_ref[0].astype(jnp.float32) + c, 0.0)


def _final_norm(y, stats, g, be, rows=64):
    bsz, cout = y.shape[0], y.shape[1]
    return pl.pallas_call(
        _final_body,
        grid=(bsz, YS // rows),
        in_specs=[
            pl.BlockSpec((1, cout, rows * XS), lambda i, t: (i, 0, t)),
            pl.BlockSpec((1, 2, cout), lambda i, t: (i, 0, 0)),
            pl.BlockSpec((cout, 1), lambda i, t: (0, 0)),
            pl.BlockSpec((cout, 1), lambda i, t: (0, 0)),
        ],
        out_specs=pl.BlockSpec((1, cout, rows * XS), lambda i, t: (i, 0, t)),
        out_shape=jax.ShapeDtypeStruct((bsz, cout, NCELL), jnp.float32),
    )(y, stats, g, be)


def _w3(W):
    return W.transpose(3, 0, 2, 1).reshape(
        3, W.shape[0], 3 * W.shape[1]).astype(jnp.bfloat16)


def kernel(radar_points_list, W1, b1, g1, be1, W2, b2, g2, be2,
           W3, b3, g3, be3, W4, b4, g4, be4):
    grid = _voxelize(radar_points_list)
    bsz = grid.shape[0]
    dummy_stats = jnp.zeros((bsz, 2, 5), jnp.float32)
    dummy_gb = jnp.zeros((5, 1), jnp.float32)
    h, s = _conv_layer(grid, dummy_stats, _w3(W1), b1[:, None],
                       dummy_gb, dummy_gb, norm_in=False)
    for (W, b, g, be, gp, bep) in ((W2, b2, g2, be2, g1, be1),
                                   (W3, b3, g3, be3, g2, be2),
                                   (W4, b4, g4, be4, g3, be3)):
        h, s_next = _conv_layer(h, s, _w3(W), b[:, None],
                                gp[:, None], bep[:, None], norm_in=True)
        s = s_next
    out = _final_norm(h, s, g4[:, None], be4[:, None])
    return out.reshape(out.shape[0], out.shape[1], YS, XS)
